# Initial kernel scaffold; baseline (speedup 1.0000x reference)
import jax
import jax.numpy as jnp
from jax.experimental import pallas as pl


def kernel(cell_feat, net_feat, pin_feat, pin_cell_idx, pin_net_idx, father_src, father_dst, net_net_pair, params):
    raise NotImplementedError("write your pallas kernel here")


# speedup vs baseline: 3.5143x; 3.5143x over previous
import functools

import jax
import jax.numpy as jnp
from jax import lax
from jax.experimental import pallas as pl
from jax.experimental.pallas import tpu as pltpu
from jax.experimental.pallas import tpu_sc as plsc

F32 = jnp.float32
I32 = jnp.int32
LOG2 = 0.6931471805599453

N_CELL, N_NET, N_PIN, N_NN = 50000, 10000, 800000, 40000
CP, NPAD = 50048, 10112
CSL, NSL = 3128, 632
NNP = 40960
NNF = 49152
NPP = 819200
PINR = NPP // 128
UF = NPAD * 8
SF = CP * 2

_mesh = plsc.VectorSubcoreMesh(core_axis_name="c", subcore_axis_name="s")


def _ssp(x):
    return jnp.maximum(x, 0.0) + jnp.log1p(jnp.exp(-jnp.abs(x))) - LOG2


def _softplus(x):
    return jnp.maximum(x, 0.0) + jnp.log1p(jnp.exp(-jnp.abs(x)))


def _rs(c):
    return lax.rsqrt(jnp.maximum(c, 1.0))


def _wid_axes():
    return lax.axis_index("c"), lax.axis_index("s")


def _fire_drain(descs):
    for d in descs:
        d.wait()


@functools.partial(
    pl.kernel,
    out_type=[
        jax.ShapeDtypeStruct((CP,), F32),
        jax.ShapeDtypeStruct((NPAD,), F32),
        jax.ShapeDtypeStruct((NPAD,), F32),
        jax.ShapeDtypeStruct((NPAD,), F32),
    ],
    mesh=_mesh,
    compiler_params=pltpu.CompilerParams(use_tc_tiling_on_sc=False),
    scratch_types=[
        pltpu.VMEM_SHARED((CP,), F32),
        pltpu.VMEM_SHARED((NPAD,), F32),
        pltpu.VMEM_SHARED((NPAD,), F32),
        pltpu.VMEM_SHARED((NPAD,), F32),
        pltpu.VMEM((16, 128), I32),
        pltpu.VMEM((16, 128), F32),
        pltpu.VMEM((CSL,), F32),
        pltpu.SemaphoreType.DMA,
    ],
)
def _k0(pci2, pni2, fs2, fd2, ones_h, z1, out_cc, out_cn, out_cf, out_cd,
        hc_s, hn_s, hf_s, hd_s, idx_v, ones_v, zb, sem):
    cid, sid = _wid_axes()
    pltpu.sync_copy(ones_h, ones_v)
    pltpu.sync_copy(z1, zb)
    pltpu.sync_copy(zb, hc_s.at[pl.ds(sid * CSL, CSL)])
    pltpu.sync_copy(zb.at[pl.ds(0, NSL)], hn_s.at[pl.ds(sid * NSL, NSL)])
    pltpu.sync_copy(zb.at[pl.ds(0, NSL)], hf_s.at[pl.ds(sid * NSL, NSL)])
    pltpu.sync_copy(zb.at[pl.ds(0, NSL)], hd_s.at[pl.ds(sid * NSL, NSL)])
    plsc.subcore_barrier()

    def pin_hist(src2d, hist):
        def body(k, _):
            c = sid + 16 * k
            pltpu.sync_copy(src2d.at[pl.ds(c * 16, 16)], idx_v)
            _fire_drain([
                pltpu.async_copy(ones_v.at[j], hist.at[idx_v.at[j]], sem,
                                 add=True)
                for j in range(16)
            ])
            return 0

        lax.fori_loop(0, 25, body, 0)

    @pl.when(cid == 0)
    def _():
        pin_hist(pci2, hc_s)

    @pl.when(cid == 1)
    def _():
        pin_hist(pni2, hn_s)

        def body2(k, _):
            c = sid + 16 * k

            @pl.when(c < 48)
            def _():
                r0 = c * 8
                pltpu.sync_copy(fs2.at[pl.ds(r0, 8)], idx_v.at[pl.ds(0, 8)])
                _fire_drain([
                    pltpu.async_copy(ones_v.at[j], hf_s.at[idx_v.at[j]], sem,
                                     add=True)
                    for j in range(8)
                ])
                pltpu.sync_copy(fd2.at[pl.ds(r0, 8)], idx_v.at[pl.ds(0, 8)])
                _fire_drain([
                    pltpu.async_copy(ones_v.at[j], hd_s.at[idx_v.at[j]], sem,
                                     add=True)
                    for j in range(8)
                ])
            return 0

        lax.fori_loop(0, 3, body2, 0)

    plsc.subcore_barrier()

    @pl.when(cid == 0)
    def _():
        pltpu.sync_copy(hc_s.at[pl.ds(sid * CSL, CSL)], zb)
        pltpu.sync_copy(zb, out_cc.at[pl.ds(sid * CSL, CSL)])

    @pl.when(cid == 1)
    def _():
        s0 = sid * NSL
        pltpu.sync_copy(hn_s.at[pl.ds(s0, NSL)], zb.at[pl.ds(0, NSL)])
        pltpu.sync_copy(zb.at[pl.ds(0, NSL)], out_cn.at[pl.ds(s0, NSL)])
        pltpu.sync_copy(hf_s.at[pl.ds(s0, NSL)], zb.at[pl.ds(0, NSL)])
        pltpu.sync_copy(zb.at[pl.ds(0, NSL)], out_cf.at[pl.ds(s0, NSL)])
        pltpu.sync_copy(hd_s.at[pl.ds(s0, NSL)], zb.at[pl.ds(0, NSL)])
        pltpu.sync_copy(zb.at[pl.ds(0, NSL)], out_cd.at[pl.ds(s0, NSL)])


@functools.partial(
    pl.kernel,
    out_type=[jax.ShapeDtypeStruct((2 * NPAD, 64), F32)],
    mesh=_mesh,
    compiler_params=pltpu.CompilerParams(use_tc_tiling_on_sc=False),
    scratch_types=[
        pltpu.VMEM_SHARED((NPAD, 64), F32),
        pltpu.VMEM((8, 128), I32),
        pltpu.VMEM((8, 128), I32),
        pltpu.VMEM((1024, 64), F32),
        pltpu.SemaphoreType.DMA,
        pltpu.SemaphoreType.DMA,
    ],
)
def _k1(hn2, fsd2, z64, out2,
        acc_s, idxg, idxs, rows, semg, sems):
    cid, sid = _wid_axes()
    s0 = sid * NSL

    stg = rows.at[pl.ds(0, NSL), :]
    pltpu.sync_copy(z64, stg)
    pltpu.sync_copy(stg, acc_s.at[pl.ds(s0, NSL)])
    plsc.subcore_barrier()

    def body(k, _):
        c = sid + 16 * k
        r0 = c * 8
        pltpu.sync_copy(fsd2.at[pl.ds(cid * 384 + r0, 8)], idxg)
        pltpu.sync_copy(fsd2.at[pl.ds((1 - cid) * 384 + r0, 8)], idxs)

        off = cid * NPAD

        @plsc.parallel_loop(0, 64, 1, unroll=8)
        def _(i):
            r = i // 8
            cc = (i % 8) * 16
            idxg[r, pl.ds(cc, 16)] = idxg[r, pl.ds(cc, 16)] + off

        _fire_drain([
            pltpu.async_copy(hn2.at[idxg.at[j]],
                             rows.at[pl.ds(j * 128, 128)], semg)
            for j in range(8)
        ])
        _fire_drain([
            pltpu.async_copy(rows.at[pl.ds(j * 128, 128)],
                             acc_s.at[idxs.at[j]], sems, add=True)
            for j in range(8)
        ])
        return 0

    lax.fori_loop(0, 3, body, 0)
    plsc.subcore_barrier()

    pltpu.sync_copy(acc_s.at[pl.ds(s0, NSL)], stg)
    pltpu.sync_copy(stg, out2.at[pl.ds(cid * NPAD + s0, NSL)])


@functools.partial(
    pl.kernel,
    out_type=[
        jax.ShapeDtypeStruct((NPAD, 32), F32),
        jax.ShapeDtypeStruct((NPAD, 32), F32),
    ],
    mesh=_mesh,
    compiler_params=pltpu.CompilerParams(use_tc_tiling_on_sc=False),
    scratch_types=[
        pltpu.VMEM_SHARED((NPAD, 32), F32),
        pltpu.VMEM((16, 128), I32),
        pltpu.VMEM((16, 128), I32),
        pltpu.VMEM((2048, 32), F32),
        pltpu.SemaphoreType.DMA,
        pltpu.SemaphoreType.DMA,
    ],
)
def _k2(hc_lo, hc_hi, pci2, pni2, z2d, out_lo, out_hi,
        acc_s, idxg, idxs, rows, semg, sems):
    cid, sid = _wid_axes()
    s0 = sid * NSL

    stgz = rows.at[pl.ds(0, NSL), :]
    pltpu.sync_copy(z2d.at[pl.ds(0, NSL)], stgz)
    pltpu.sync_copy(stgz, acc_s.at[pl.ds(s0, NSL)])
    plsc.subcore_barrier()

    def body(k, _):
        c = sid + 16 * k
        r0 = c * 16
        pltpu.sync_copy(pci2.at[pl.ds(r0, 16)], idxg)
        pltpu.sync_copy(pni2.at[pl.ds(r0, 16)], idxs)

        @pl.when(cid == 0)
        def _():
            _fire_drain([
                pltpu.async_copy(hc_lo.at[idxg.at[j]],
                                 rows.at[pl.ds(j * 128, 128)], semg)
                for j in range(16)
            ])

        @pl.when(cid == 1)
        def _():
            _fire_drain([
                pltpu.async_copy(hc_hi.at[idxg.at[j]],
                                 rows.at[pl.ds(j * 128, 128)], semg)
                for j in range(16)
            ])

        _fire_drain([
            pltpu.async_copy(rows.at[pl.ds(j * 128, 128)],
                             acc_s.at[idxs.at[j]], sems, add=True)
            for j in range(16)
        ])
        return 0

    lax.fori_loop(0, 25, body, 0)
    plsc.subcore_barrier()

    stgo = rows.at[pl.ds(0, NSL), :]
    pltpu.sync_copy(acc_s.at[pl.ds(s0, NSL)], stgo)

    @pl.when(cid == 0)
    def _():
        pltpu.sync_copy(stgo, out_lo.at[pl.ds(s0, NSL)])

    @pl.when(cid == 1)
    def _():
        pltpu.sync_copy(stgo, out_hi.at[pl.ds(s0, NSL)])


@functools.partial(
    pl.kernel,
    out_type=[jax.ShapeDtypeStruct((CP, 16), F32) for _ in range(4)],
    mesh=_mesh,
    compiler_params=pltpu.CompilerParams(use_tc_tiling_on_sc=False),
    scratch_types=[
        pltpu.VMEM_SHARED((CP, 16), F32),
        pltpu.VMEM((16, 128), I32),
        pltpu.VMEM((16, 128), I32),
        pltpu.VMEM((2048, 16), F32),
        pltpu.VMEM((2048, 16), F32),
        pltpu.SemaphoreType.DMA,
        pltpu.SemaphoreType.DMA,
    ],
)
def _k3(hv0, hv1, hv2, hv3, he0, he1, he2, he3, pci2, pni2, z16,
        out0, out1, out2, out3,
        acc_s, idxg, idxs, rows, heb, semg, sems):
    cid, sid = _wid_axes()
    c0 = sid * CSL

    def one_pass(hvq, heq, outq):
        for off, n in ((0, 2048), (2048, 1080)):
            stgz = rows.at[pl.ds(0, n), :]
            pltpu.sync_copy(z16.at[pl.ds(off, n)], stgz)
            pltpu.sync_copy(stgz, acc_s.at[pl.ds(c0 + off, n)])
        plsc.subcore_barrier()

        def body(k, _):
            c = sid + 16 * k
            r0 = c * 16
            pltpu.sync_copy(pni2.at[pl.ds(r0, 16)], idxg)
            pltpu.sync_copy(pci2.at[pl.ds(r0, 16)], idxs)
            pltpu.sync_copy(heq.at[pl.ds(c * 2048, 2048), :], heb)
            _fire_drain([
                pltpu.async_copy(hvq.at[idxg.at[j]],
                                 rows.at[pl.ds(j * 128, 128)], semg)
                for j in range(16)
            ])

            @plsc.parallel_loop(0, 2048, 1, unroll=8)
            def _(r):
                heb[r, pl.ds(0, 16)] = (heb[r, pl.ds(0, 16)] *
                                        rows[r, pl.ds(0, 16)])

            _fire_drain([
                pltpu.async_copy(heb.at[pl.ds(j * 128, 128)],
                                 acc_s.at[idxs.at[j]], sems, add=True)
                for j in range(16)
            ])
            return 0

        lax.fori_loop(0, 25, body, 0)
        plsc.subcore_barrier()

        for off, n in ((0, 2048), (2048, 1080)):
            stgo = rows.at[pl.ds(0, n), :]
            pltpu.sync_copy(acc_s.at[pl.ds(c0 + off, n)], stgo)
            pltpu.sync_copy(stgo, outq.at[pl.ds(c0 + off, n)])
        plsc.subcore_barrier()

    @pl.when(cid == 0)
    def _():
        one_pass(hv0, he0, out0)
        one_pass(hv1, he1, out1)

    @pl.when(cid == 1)
    def _():
        one_pass(hv2, he2, out2)
        one_pass(hv3, he3, out3)


@functools.partial(
    pl.kernel,
    out_type=[
        jax.ShapeDtypeStruct((320, 128), F32),
        jax.ShapeDtypeStruct((320, 128), F32),
        jax.ShapeDtypeStruct((PINR, 128), F32),
        jax.ShapeDtypeStruct((PINR, 128), F32),
    ],
    mesh=_mesh,
    compiler_params=pltpu.CompilerParams(use_tc_tiling_on_sc=False),
    scratch_types=[
        pltpu.VMEM_SHARED((UF,), F32),
        pltpu.VMEM_SHARED((SF,), F32),
        pltpu.VMEM((16, 128), I32),
        pltpu.VMEM((16, 128), I32),
        pltpu.VMEM((16, 128), I32),
        pltpu.VMEM((16, 128), I32),
        pltpu.VMEM((16, 128), I32),
        pltpu.VMEM((16, 128), I32),
        pltpu.VMEM((16, 128), F32),
        pltpu.VMEM((16, 128), F32),
        pltpu.VMEM((16, 128), F32),
        pltpu.VMEM((16, 128), F32),
        pltpu.VMEM((SF // 16,), F32),
        pltpu.SemaphoreType.DMA,
    ],
)
def _k4(uf, scf, nn0_2, nn1_2, pni2, pci2, nd0, na0, pd0, pa0,
        u_s, sc_s, ia, ib, f0, f1, f2, f3, g0, g1, g2, g3, stb, sem):
    cid, sid = _wid_axes()
    pltpu.sync_copy(uf.at[pl.ds(sid * (UF // 16), UF // 16)],
                    stb.at[pl.ds(0, UF // 16)])
    pltpu.sync_copy(stb.at[pl.ds(0, UF // 16)],
                    u_s.at[pl.ds(sid * (UF // 16), UF // 16)])
    pltpu.sync_copy(scf.at[pl.ds(sid * (SF // 16), SF // 16)], stb)
    pltpu.sync_copy(stb, sc_s.at[pl.ds(sid * (SF // 16), SF // 16)])
    plsc.subcore_barrier()

    @pl.when(cid == 0)
    def _():
        def nbody(k, _):
            c = sid + 16 * k

            @pl.when(c < 40)
            def _():
                r0 = c * 8
                pltpu.sync_copy(nn0_2.at[pl.ds(r0, 8)], ia.at[pl.ds(0, 8)])
                pltpu.sync_copy(nn1_2.at[pl.ds(r0, 8)], ib.at[pl.ds(0, 8)])

                @plsc.parallel_loop(0, 64, 1, unroll=8)
                def _(i):
                    r = i // 8
                    cc = (i % 8) * 16
                    va = ia[r, pl.ds(cc, 16)] * 8
                    vb = ib[r, pl.ds(cc, 16)] * 8
                    f0[r, pl.ds(cc, 16)] = va
                    f1[r, pl.ds(cc, 16)] = vb + 1
                    f2[r, pl.ds(cc, 16)] = va + 2
                    f3[r, pl.ds(cc, 16)] = vb + 3

                _fire_drain(
                    [pltpu.async_copy(u_s.at[f0.at[j]], g0.at[j], sem)
                     for j in range(8)] +
                    [pltpu.async_copy(u_s.at[f1.at[j]], g1.at[j], sem)
                     for j in range(8)] +
                    [pltpu.async_copy(u_s.at[f2.at[j]], g2.at[j], sem)
                     for j in range(8)] +
                    [pltpu.async_copy(u_s.at[f3.at[j]], g3.at[j], sem)
                     for j in range(8)])

                @plsc.parallel_loop(0, 64, 1, unroll=8)
                def _(i):
                    r = i // 8
                    cc = (i % 8) * 16
                    g0[r, pl.ds(cc, 16)] = (g0[r, pl.ds(cc, 16)] +
                                            g1[r, pl.ds(cc, 16)])
                    g2[r, pl.ds(cc, 16)] = (g2[r, pl.ds(cc, 16)] +
                                            g3[r, pl.ds(cc, 16)])

                pltpu.sync_copy(g0.at[pl.ds(0, 8)], nd0.at[pl.ds(r0, 8)])
                pltpu.sync_copy(g2.at[pl.ds(0, 8)], na0.at[pl.ds(r0, 8)])
            return 0

        lax.fori_loop(0, 3, nbody, 0)

    def pbody(k, _):
        ci = sid + 16 * k

        @pl.when(ci < 200)
        def _():
            c = 2 * ci + cid
            r0 = c * 16
            pltpu.sync_copy(pni2.at[pl.ds(r0, 16)], ia)
            pltpu.sync_copy(pci2.at[pl.ds(r0, 16)], ib)

            @plsc.parallel_loop(0, 128, 1, unroll=8)
            def _(i):
                r = i // 8
                cc = (i % 8) * 16
                va = ia[r, pl.ds(cc, 16)] * 8
                vb = ib[r, pl.ds(cc, 16)] * 2
                f0[r, pl.ds(cc, 16)] = va + 4
                f1[r, pl.ds(cc, 16)] = va + 5
                f2[r, pl.ds(cc, 16)] = vb
                f3[r, pl.ds(cc, 16)] = vb + 1

            _fire_drain(
                [pltpu.async_copy(u_s.at[f0.at[j]], g0.at[j], sem)
                 for j in range(16)] +
                [pltpu.async_copy(u_s.at[f1.at[j]], g1.at[j], sem)
                 for j in range(16)] +
                [pltpu.async_copy(sc_s.at[f2.at[j]], g2.at[j], sem)
                 for j in range(16)] +
                [pltpu.async_copy(sc_s.at[f3.at[j]], g3.at[j], sem)
                 for j in range(16)])

            @plsc.parallel_loop(0, 128, 1, unroll=8)
            def _(i):
                r = i // 8
                cc = (i % 8) * 16
                g0[r, pl.ds(cc, 16)] = (g0[r, pl.ds(cc, 16)] +
                                        g2[r, pl.ds(cc, 16)])
                g1[r, pl.ds(cc, 16)] = (g1[r, pl.ds(cc, 16)] +
                                        g3[r, pl.ds(cc, 16)])

            pltpu.sync_copy(g0, pd0.at[pl.ds(r0, 16)])
            pltpu.sync_copy(g1, pa0.at[pl.ds(r0, 16)])
        return 0

    lax.fori_loop(0, 13, pbody, 0)


def _t1a_body(x_ref, cnt_ref, w_ref, b_ref, lo_ref, hi_ref):
    h = jnp.tanh(jnp.dot(x_ref[...], w_ref[...],
                         preferred_element_type=F32) + b_ref[...])
    h = h * _rs(cnt_ref[...])
    lo_ref[...] = h[:, :32]
    hi_ref[...] = h[:, 32:]


def _t1b_body(x_ref, cnt3_ref, w_ref, b_ref, cw_ref, cb_ref,
              hs_ref, hd_ref, lo_ref):
    hn = jnp.tanh(jnp.dot(x_ref[...], w_ref[...],
                          preferred_element_type=F32) + b_ref[...])
    cnt3 = cnt3_ref[...]
    hs_ref[...] = hn * _rs(cnt3[:, 2:3])
    hd_ref[...] = hn * _rs(cnt3[:, 1:2])
    hv = jnp.dot(hn, cw_ref[...], preferred_element_type=F32) + cb_ref[...]
    lo_ref[...] = hv


def _t1c_body(x_ref, w1_ref, b1_ref, w2_ref, b2_ref, w3_ref, b3_ref, wsp_ref,
              q0_ref, q1_ref, q2_ref, q3_ref, sp_ref):
    hp = jnp.tanh(jnp.dot(x_ref[...], w1_ref[...],
                          preferred_element_type=F32) + b1_ref[...])
    t = _ssp(jnp.dot(hp, w2_ref[...], preferred_element_type=F32) + b2_ref[...])
    he = _ssp(jnp.dot(t, w3_ref[...], preferred_element_type=F32) + b3_ref[...])
    q0_ref[...] = he[:, :16]
    q1_ref[...] = he[:, 16:32]
    q2_ref[...] = he[:, 32:48]
    q3_ref[...] = he[:, 48:]
    sp_ref[...] = jnp.dot(hp, wsp_ref[...], preferred_element_type=F32)


def _t2a_body(a1l_ref, a1h_ref, af_ref, as_ref, cnt3_ref,
              wp_ref, bp_ref, wf_ref, bf_ref, ws_ref, bs_ref, w6_ref, b6_ref,
              u_ref):
    cnt3 = cnt3_ref[...]
    a1 = jnp.concatenate([a1l_ref[...], a1h_ref[...]], axis=-1)
    op = jnp.dot(a1 * _rs(cnt3[:, 0:1]), wp_ref[...],
                 preferred_element_type=F32) + bp_ref[...]
    of = jnp.dot(af_ref[...] * _rs(cnt3[:, 1:2]), wf_ref[...],
                 preferred_element_type=F32) + bf_ref[...]
    os_ = jnp.dot(as_ref[...] * _rs(cnt3[:, 2:3]), ws_ref[...],
                  preferred_element_type=F32) + bs_ref[...]
    h = jnp.maximum(jnp.maximum(op, of), os_)
    u_ref[...] = (jnp.dot(h, w6_ref[...], preferred_element_type=F32) +
                  b6_ref[...])


def _t2b_body(a40_ref, a41_ref, a42_ref, a43_ref, wo_ref, bo_ref, wsc_ref,
              sc_ref):
    a4 = jnp.concatenate([a40_ref[...], a41_ref[...], a42_ref[...],
                          a43_ref[...]], axis=-1)
    h = _ssp(jnp.dot(a4, wo_ref[...], preferred_element_type=F32) + bo_ref[...])
    sc_ref[...] = jnp.dot(h, wsc_ref[...], preferred_element_type=F32)


def _t3_body(pd_ref, pa_ref, sd_ref, sa_ref, dis_ref, ang_ref):
    dis_ref[...] = _softplus(pd_ref[...] + sd_ref[...])
    ang_ref[...] = pa_ref[...] + sa_ref[...]


def _t3b_body(nd_ref, dis_ref):
    dis_ref[...] = _softplus(nd_ref[...])


def _full(shape):
    nd = len(shape)
    return pl.BlockSpec(shape, lambda *_: (0,) * nd)


def kernel(cell_feat, net_feat, pin_feat, pin_cell_idx, pin_net_idx,
           father_src, father_dst, net_net_pair, params):
    p = params
    pci = pin_cell_idx.astype(I32)
    pni = pin_net_idx.astype(I32)
    fs = father_src.astype(I32)
    fd = father_dst.astype(I32)
    nn0 = net_net_pair[:, 0].astype(I32)
    nn1 = net_net_pair[:, 1].astype(I32)

    dead_n = 10048 + (jnp.arange(NNP - N_NN, dtype=I32) % 64)
    dead_f = 10048 + (jnp.arange(NNF - N_NN, dtype=I32) % 64)
    dead_np = 10048 + (jnp.arange(NPP - N_PIN, dtype=I32) % 64)
    dead_cp = 50000 + (jnp.arange(NPP - N_PIN, dtype=I32) % 48)
    pci2 = jnp.concatenate([pci, dead_cp]).reshape(PINR, 128)
    pni2 = jnp.concatenate([pni, dead_np]).reshape(PINR, 128)
    fs2 = jnp.concatenate([fs, dead_f]).reshape(384, 128)
    fd2 = jnp.concatenate([fd, dead_f]).reshape(384, 128)
    nn0_2 = jnp.concatenate([nn0, dead_n]).reshape(320, 128)
    nn1_2 = jnp.concatenate([nn1, dead_n]).reshape(320, 128)

    ones_h = jnp.ones((16, 128), F32)
    z1 = jnp.zeros((CSL,), F32)
    z2d = jnp.zeros((CSL, 32), F32)
    z16 = jnp.zeros((CSL, 16), F32)
    z64 = jnp.zeros((NSL, 64), F32)

    cell_p = jnp.pad(cell_feat, ((0, CP - N_CELL), (0, 0)))
    net_p = jnp.pad(net_feat, ((0, NPAD - N_NET), (0, 0)))
    pin_p = jnp.pad(pin_feat, ((0, NPP - N_PIN), (0, 0)))

    wd, wa = p["net_dis"]["W"][:, 0], p["net_angle"]["W"][:, 0]
    wpd, wpa = p["pin_dis"]["W"][:, 0], p["pin_angle"]["W"][:, 0]
    zc = jnp.zeros((64,), F32)
    w6 = jnp.stack([wd[:64], wd[64:], wa[:64], wa[64:], wpd[:64], wpa[:64],
                    zc, zc], axis=-1)
    e = jnp.eye(8, dtype=F32)
    b6 = (e[0] * p["net_dis"]["b"][0] + e[2] * p["net_angle"]["b"][0] +
          e[4] * p["pin_dis"]["b"][0] + e[5] * p["pin_angle"]["b"][0])
    b6 = b6.reshape(1, 8)
    wsp = jnp.stack([wpd[64:80], wpa[64:80]], axis=-1)
    wsc = jnp.stack([wpd[80:], wpa[80:]], axis=-1)

    cc, cn, cf, cd = _k0(pci2, pni2, fs2, fd2, ones_h, z1)
    cnt3 = jnp.stack([cn, cd, cf], axis=-1)

    hc_lo, hc_hi = pl.pallas_call(
        _t1a_body,
        grid=(23,),
        in_specs=[
            pl.BlockSpec((2176, 16), lambda i: (i, 0)),
            pl.BlockSpec((2176, 1), lambda i: (i, 0)),
            _full((16, 64)),
            _full((1, 64)),
        ],
        out_specs=[
            pl.BlockSpec((2176, 32), lambda i: (i, 0)),
            pl.BlockSpec((2176, 32), lambda i: (i, 0)),
        ],
        out_shape=[
            jax.ShapeDtypeStruct((CP, 32), F32),
            jax.ShapeDtypeStruct((CP, 32), F32),
        ],
    )(cell_p, cc.reshape(CP, 1), p["cell_lin"]["W"],
      p["cell_lin"]["b"].reshape(1, 64))

    hn_src, hn_dst, hv = pl.pallas_call(
        _t1b_body,
        grid=(8,),
        in_specs=[
            pl.BlockSpec((1264, 8), lambda i: (i, 0)),
            pl.BlockSpec((1264, 3), lambda i: (i, 0)),
            _full((8, 64)),
            _full((1, 64)),
            _full((64, 64)),
            _full((1, 64)),
        ],
        out_specs=[
            pl.BlockSpec((1264, 64), lambda i: (i, 0)),
            pl.BlockSpec((1264, 64), lambda i: (i, 0)),
            pl.BlockSpec((1264, 64), lambda i: (i, 0)),
        ],
        out_shape=[
            jax.ShapeDtypeStruct((NPAD, 64), F32),
            jax.ShapeDtypeStruct((NPAD, 64), F32),
            jax.ShapeDtypeStruct((NPAD, 64), F32),
        ],
    )(net_p, cnt3, p["net_lin"]["W"], p["net_lin"]["b"].reshape(1, 64),
      p["cf_node"]["W"], p["cf_node"]["b"].reshape(1, 64))

    he0, he1, he2, he3, s_pin = pl.pallas_call(
        _t1c_body,
        grid=(400,),
        in_specs=[
            pl.BlockSpec((2048, 8), lambda i: (i, 0)),
            _full((8, 16)),
            _full((1, 16)),
            _full((16, 64)),
            _full((1, 64)),
            _full((64, 64)),
            _full((1, 64)),
            _full((16, 2)),
        ],
        out_specs=[
            pl.BlockSpec((2048, 16), lambda i: (i, 0)),
            pl.BlockSpec((2048, 16), lambda i: (i, 0)),
            pl.BlockSpec((2048, 16), lambda i: (i, 0)),
            pl.BlockSpec((2048, 16), lambda i: (i, 0)),
            pl.BlockSpec((2048, 2), lambda i: (i, 0)),
        ],
        out_shape=[
            jax.ShapeDtypeStruct((NPP, 16), F32),
            jax.ShapeDtypeStruct((NPP, 16), F32),
            jax.ShapeDtypeStruct((NPP, 16), F32),
            jax.ShapeDtypeStruct((NPP, 16), F32),
            jax.ShapeDtypeStruct((NPP, 2), F32),
        ],
    )(pin_p, p["pin_lin"]["W"], p["pin_lin"]["b"].reshape(1, 16),
      p["cf_edge1"]["W"], p["cf_edge1"]["b"].reshape(1, 64),
      p["cf_edge2"]["W"], p["cf_edge2"]["b"].reshape(1, 64), wsp)

    fsd2 = jnp.concatenate([fs2, fd2], axis=0)
    hn2 = jnp.concatenate([hn_src, hn_dst], axis=0)
    out2 = _k1(hn2, fsd2, z64)[0]
    accf, accs = out2[:NPAD], out2[NPAD:]
    acc1_lo, acc1_hi = _k2(hc_lo, hc_hi, pci2, pni2, z2d)
    hv0, hv1, hv2, hv3 = (hv[:, :16], hv[:, 16:32], hv[:, 32:48], hv[:, 48:])
    a40, a41, a42, a43 = _k3(hv0, hv1, hv2, hv3, he0, he1, he2, he3,
                             pci2, pni2, z16)

    u = pl.pallas_call(
        _t2a_body,
        grid=(4,),
        in_specs=[
            pl.BlockSpec((2528, 32), lambda i: (i, 0)),
            pl.BlockSpec((2528, 32), lambda i: (i, 0)),
            pl.BlockSpec((2528, 64), lambda i: (i, 0)),
            pl.BlockSpec((2528, 64), lambda i: (i, 0)),
            pl.BlockSpec((2528, 3), lambda i: (i, 0)),
            _full((64, 64)), _full((1, 64)),
            _full((64, 64)), _full((1, 64)),
            _full((64, 64)), _full((1, 64)),
            _full((64, 8)), _full((1, 8)),
        ],
        out_specs=[pl.BlockSpec((2528, 8), lambda i: (i, 0))],
        out_shape=[jax.ShapeDtypeStruct((NPAD, 8), F32)],
    )(acc1_lo, acc1_hi, accf, accs, cnt3,
      p["gc_pins"]["W"], p["gc_pins"]["b"].reshape(1, 64),
      p["gc_father"]["W"], p["gc_father"]["b"].reshape(1, 64),
      p["gc_son"]["W"], p["gc_son"]["b"].reshape(1, 64),
      w6, b6)[0]

    s_cell = pl.pallas_call(
        _t2b_body,
        grid=(23,),
        in_specs=[
            pl.BlockSpec((2176, 16), lambda i: (i, 0)),
            pl.BlockSpec((2176, 16), lambda i: (i, 0)),
            pl.BlockSpec((2176, 16), lambda i: (i, 0)),
            pl.BlockSpec((2176, 16), lambda i: (i, 0)),
            _full((64, 64)), _full((1, 64)), _full((64, 2)),
        ],
        out_specs=[pl.BlockSpec((2176, 2), lambda i: (i, 0))],
        out_shape=[jax.ShapeDtypeStruct((CP, 2), F32)],
    )(a40, a41, a42, a43, p["cf_out"]["W"], p["cf_out"]["b"].reshape(1, 64),
      wsc)[0]

    nd0, na0, pd0, pa0 = _k4(u.reshape(UF), s_cell.reshape(SF),
                             nn0_2, nn1_2, pni2, pci2)

    net_dis2 = pl.pallas_call(
        _t3b_body,
        in_specs=[_full((320, 128))],
        out_specs=[_full((320, 128))],
        out_shape=[jax.ShapeDtypeStruct((320, 128), F32)],
    )(nd0)[0]

    sd = s_pin[:, 0].reshape(PINR, 128)
    sa = s_pin[:, 1].reshape(PINR, 128)
    dis2, ang2 = pl.pallas_call(
        _t3_body,
        grid=(8,),
        in_specs=[pl.BlockSpec((800, 128), lambda i: (i, 0))] * 4,
        out_specs=[pl.BlockSpec((800, 128), lambda i: (i, 0))] * 2,
        out_shape=[
            jax.ShapeDtypeStruct((PINR, 128), F32),
            jax.ShapeDtypeStruct((PINR, 128), F32),
        ],
    )(pd0, pa0, sd, sa)

    return (net_dis2.reshape(-1)[:N_NN], na0.reshape(-1)[:N_NN],
            dis2.reshape(-1)[:N_PIN], ang2.reshape(-1)[:N_PIN])


# speedup vs baseline: 3.6434x; 1.0367x over previous
import functools

import jax
import jax.numpy as jnp
from jax import lax
from jax.experimental import pallas as pl
from jax.experimental.pallas import tpu as pltpu
from jax.experimental.pallas import tpu_sc as plsc

F32 = jnp.float32
I32 = jnp.int32
LOG2 = 0.6931471805599453

N_CELL, N_NET, N_PIN, N_NN = 50000, 10000, 800000, 40000
CP, NPAD = 50048, 10112
CSL, NSL = 3128, 632
NNP = 40960
NNF = 49152
NPP = 819200
PINR = NPP // 128
UF = NPAD * 8
SF = CP * 2

_mesh = plsc.VectorSubcoreMesh(core_axis_name="c", subcore_axis_name="s")


def _ssp(x):
    return jnp.maximum(x, 0.0) + jnp.log1p(jnp.exp(-jnp.abs(x))) - LOG2


def _softplus(x):
    return jnp.maximum(x, 0.0) + jnp.log1p(jnp.exp(-jnp.abs(x)))


def _rs(c):
    return lax.rsqrt(jnp.maximum(c, 1.0))


def _wid_axes():
    return lax.axis_index("c"), lax.axis_index("s")


def _fire_drain(descs):
    for d in descs:
        d.wait()


@functools.partial(
    pl.kernel,
    out_type=[
        jax.ShapeDtypeStruct((CP,), F32),
        jax.ShapeDtypeStruct((NPAD,), F32),
        jax.ShapeDtypeStruct((NPAD,), F32),
        jax.ShapeDtypeStruct((NPAD,), F32),
    ],
    mesh=_mesh,
    compiler_params=pltpu.CompilerParams(use_tc_tiling_on_sc=False),
    scratch_types=[
        pltpu.VMEM_SHARED((CP,), F32),
        pltpu.VMEM_SHARED((NPAD,), F32),
        pltpu.VMEM_SHARED((NPAD,), F32),
        pltpu.VMEM_SHARED((NPAD,), F32),
        pltpu.VMEM((16, 128), I32),
        pltpu.VMEM((16, 128), F32),
        pltpu.VMEM((CSL,), F32),
        pltpu.SemaphoreType.DMA,
    ],
)
def _k0(pci2, pni2, fs2, fd2, ones_h, z1, out_cc, out_cn, out_cf, out_cd,
        hc_s, hn_s, hf_s, hd_s, idx_v, ones_v, zb, sem):
    cid, sid = _wid_axes()
    pltpu.sync_copy(ones_h, ones_v)
    pltpu.sync_copy(z1, zb)
    pltpu.sync_copy(zb, hc_s.at[pl.ds(sid * CSL, CSL)])
    pltpu.sync_copy(zb.at[pl.ds(0, NSL)], hn_s.at[pl.ds(sid * NSL, NSL)])
    pltpu.sync_copy(zb.at[pl.ds(0, NSL)], hf_s.at[pl.ds(sid * NSL, NSL)])
    pltpu.sync_copy(zb.at[pl.ds(0, NSL)], hd_s.at[pl.ds(sid * NSL, NSL)])
    plsc.subcore_barrier()

    def pin_hist(src2d, hist):
        def body(k, _):
            c = sid + 16 * k
            pltpu.sync_copy(src2d.at[pl.ds(c * 16, 16)], idx_v)
            _fire_drain([
                pltpu.async_copy(ones_v.at[j], hist.at[idx_v.at[j]], sem,
                                 add=True)
                for j in range(16)
            ])
            return 0

        lax.fori_loop(0, 25, body, 0)

    @pl.when(cid == 0)
    def _():
        pin_hist(pci2, hc_s)

    @pl.when(cid == 1)
    def _():
        pin_hist(pni2, hn_s)

        def body2(k, _):
            c = sid + 16 * k

            @pl.when(c < 48)
            def _():
                r0 = c * 8
                pltpu.sync_copy(fs2.at[pl.ds(r0, 8)], idx_v.at[pl.ds(0, 8)])
                _fire_drain([
                    pltpu.async_copy(ones_v.at[j], hf_s.at[idx_v.at[j]], sem,
                                     add=True)
                    for j in range(8)
                ])
                pltpu.sync_copy(fd2.at[pl.ds(r0, 8)], idx_v.at[pl.ds(0, 8)])
                _fire_drain([
                    pltpu.async_copy(ones_v.at[j], hd_s.at[idx_v.at[j]], sem,
                                     add=True)
                    for j in range(8)
                ])
            return 0

        lax.fori_loop(0, 3, body2, 0)

    plsc.subcore_barrier()

    @pl.when(cid == 0)
    def _():
        pltpu.sync_copy(hc_s.at[pl.ds(sid * CSL, CSL)], zb)
        pltpu.sync_copy(zb, out_cc.at[pl.ds(sid * CSL, CSL)])

    @pl.when(cid == 1)
    def _():
        s0 = sid * NSL
        pltpu.sync_copy(hn_s.at[pl.ds(s0, NSL)], zb.at[pl.ds(0, NSL)])
        pltpu.sync_copy(zb.at[pl.ds(0, NSL)], out_cn.at[pl.ds(s0, NSL)])
        pltpu.sync_copy(hf_s.at[pl.ds(s0, NSL)], zb.at[pl.ds(0, NSL)])
        pltpu.sync_copy(zb.at[pl.ds(0, NSL)], out_cf.at[pl.ds(s0, NSL)])
        pltpu.sync_copy(hd_s.at[pl.ds(s0, NSL)], zb.at[pl.ds(0, NSL)])
        pltpu.sync_copy(zb.at[pl.ds(0, NSL)], out_cd.at[pl.ds(s0, NSL)])


@functools.partial(
    pl.kernel,
    out_type=[jax.ShapeDtypeStruct((2 * NPAD, 64), F32)],
    mesh=_mesh,
    compiler_params=pltpu.CompilerParams(use_tc_tiling_on_sc=False),
    scratch_types=[
        pltpu.VMEM_SHARED((NPAD, 64), F32),
        pltpu.VMEM((8, 128), I32),
        pltpu.VMEM((8, 128), I32),
        pltpu.VMEM((1024, 64), F32),
        pltpu.SemaphoreType.DMA,
        pltpu.SemaphoreType.DMA,
    ],
)
def _k1(hn2, fsd2, z64, out2,
        acc_s, idxg, idxs, rows, semg, sems):
    cid, sid = _wid_axes()
    s0 = sid * NSL

    stg = rows.at[pl.ds(0, NSL), :]
    pltpu.sync_copy(z64, stg)
    pltpu.sync_copy(stg, acc_s.at[pl.ds(s0, NSL)])
    plsc.subcore_barrier()

    def body(k, _):
        c = sid + 16 * k
        r0 = c * 8
        pltpu.sync_copy(fsd2.at[pl.ds(cid * 384 + r0, 8)], idxg)
        pltpu.sync_copy(fsd2.at[pl.ds((1 - cid) * 384 + r0, 8)], idxs)

        off = cid * NPAD

        @plsc.parallel_loop(0, 64, 1, unroll=8)
        def _(i):
            r = i // 8
            cc = (i % 8) * 16
            idxg[r, pl.ds(cc, 16)] = idxg[r, pl.ds(cc, 16)] + off

        _fire_drain([
            pltpu.async_copy(hn2.at[idxg.at[j]],
                             rows.at[pl.ds(j * 128, 128)], semg)
            for j in range(8)
        ])
        _fire_drain([
            pltpu.async_copy(rows.at[pl.ds(j * 128, 128)],
                             acc_s.at[idxs.at[j]], sems, add=True)
            for j in range(8)
        ])
        return 0

    lax.fori_loop(0, 3, body, 0)
    plsc.subcore_barrier()

    pltpu.sync_copy(acc_s.at[pl.ds(s0, NSL)], stg)
    pltpu.sync_copy(stg, out2.at[pl.ds(cid * NPAD + s0, NSL)])


@functools.partial(
    pl.kernel,
    out_type=[
        jax.ShapeDtypeStruct((NPAD, 32), F32),
        jax.ShapeDtypeStruct((NPAD, 32), F32),
    ],
    mesh=_mesh,
    compiler_params=pltpu.CompilerParams(use_tc_tiling_on_sc=False),
    scratch_types=[
        pltpu.VMEM_SHARED((NPAD, 32), F32),
        pltpu.VMEM((16, 128), I32),
        pltpu.VMEM((16, 128), I32),
        pltpu.VMEM((2048, 32), F32),
        pltpu.SemaphoreType.DMA,
        pltpu.SemaphoreType.DMA,
    ],
)
def _k2(hc_lo, hc_hi, pci2, pni2, z2d, out_lo, out_hi,
        acc_s, idxg, idxs, rows, semg, sems):
    cid, sid = _wid_axes()
    s0 = sid * NSL

    stgz = rows.at[pl.ds(0, NSL), :]
    pltpu.sync_copy(z2d.at[pl.ds(0, NSL)], stgz)
    pltpu.sync_copy(stgz, acc_s.at[pl.ds(s0, NSL)])
    plsc.subcore_barrier()

    def body(k, _):
        c = sid + 16 * k
        r0 = c * 16
        pltpu.sync_copy(pci2.at[pl.ds(r0, 16)], idxg)
        pltpu.sync_copy(pni2.at[pl.ds(r0, 16)], idxs)

        @pl.when(cid == 0)
        def _():
            _fire_drain([
                pltpu.async_copy(hc_lo.at[idxg.at[j]],
                                 rows.at[pl.ds(j * 128, 128)], semg)
                for j in range(16)
            ])

        @pl.when(cid == 1)
        def _():
            _fire_drain([
                pltpu.async_copy(hc_hi.at[idxg.at[j]],
                                 rows.at[pl.ds(j * 128, 128)], semg)
                for j in range(16)
            ])

        _fire_drain([
            pltpu.async_copy(rows.at[pl.ds(j * 128, 128)],
                             acc_s.at[idxs.at[j]], sems, add=True)
            for j in range(16)
        ])
        return 0

    lax.fori_loop(0, 25, body, 0)
    plsc.subcore_barrier()

    stgo = rows.at[pl.ds(0, NSL), :]
    pltpu.sync_copy(acc_s.at[pl.ds(s0, NSL)], stgo)

    @pl.when(cid == 0)
    def _():
        pltpu.sync_copy(stgo, out_lo.at[pl.ds(s0, NSL)])

    @pl.when(cid == 1)
    def _():
        pltpu.sync_copy(stgo, out_hi.at[pl.ds(s0, NSL)])


@functools.partial(
    pl.kernel,
    out_type=[jax.ShapeDtypeStruct((CP, 16), F32) for _ in range(4)],
    mesh=_mesh,
    compiler_params=pltpu.CompilerParams(use_tc_tiling_on_sc=False),
    scratch_types=[
        pltpu.VMEM_SHARED((CP, 16), F32),
        pltpu.VMEM_SHARED((NPAD, 16), F32),
        pltpu.VMEM((8, 128), I32),
        pltpu.VMEM((8, 128), I32),
        pltpu.VMEM((8, 128), I32),
        pltpu.VMEM((1024, 16), F32),
        pltpu.VMEM((1024, 16), F32),
        pltpu.SemaphoreType.DMA,
        pltpu.SemaphoreType.DMA,
        pltpu.SemaphoreType.DMA,
    ],
)
def _k3(hv0, hv1, hv2, hv3, he0, he1, he2, he3, pci2, pni2, z16,
        out0, out1, out2, out3,
        acc_s, tab_s, idxg, idxs0, idxs1, rows, heb, semg, sems, semh):
    cid, sid = _wid_axes()
    c0 = sid * CSL
    s0 = sid * NSL

    def one_pass(hvq, heq, outq):
        stg = rows.at[pl.ds(0, NSL), :]
        pltpu.sync_copy(hvq.at[pl.ds(s0, NSL)], stg)
        pltpu.sync_copy(stg, tab_s.at[pl.ds(s0, NSL)])
        for off, n in ((0, 1024), (1024, 1024), (2048, 1024), (3072, 56)):
            stgz = rows.at[pl.ds(0, n), :]
            pltpu.sync_copy(z16.at[pl.ds(off, n)], stgz)
            pltpu.sync_copy(stgz, acc_s.at[pl.ds(c0 + off, n)])
        plsc.subcore_barrier()

        def drain_prev():
            pltpu.make_async_copy(heq.at[pl.ds(0, 1024), :], heb,
                                  sems).wait()

        pltpu.sync_copy(pci2.at[pl.ds(0, 8)], idxs1)
        for j in range(8):
            pltpu.async_copy(rows.at[pl.ds(j * 128, 128), :],
                             acc_s.at[idxs1.at[j]], sems, add=True)

        def do_chunk(k, idxs):
            c = sid + 16 * k
            r0 = c * 8
            pltpu.sync_copy(pni2.at[pl.ds(r0, 8)], idxg)
            pltpu.sync_copy(pci2.at[pl.ds(r0, 8)], idxs)
            hed = pltpu.async_copy(heq.at[pl.ds(c * 1024, 1024), :], heb,
                                   semh)
            drain_prev()
            _fire_drain([
                pltpu.async_copy(tab_s.at[idxg.at[j]],
                                 rows.at[pl.ds(j * 128, 128)], semg)
                for j in range(8)
            ])
            hed.wait()

            @plsc.parallel_loop(0, 1024, 1, unroll=8)
            def _(q):
                rows[q, pl.ds(0, 16)] = (heb[q, pl.ds(0, 16)] *
                                         rows[q, pl.ds(0, 16)])

            for j in range(8):
                pltpu.async_copy(rows.at[pl.ds(j * 128, 128), :],
                                 acc_s.at[idxs.at[j]], sems, add=True)

        def body(j, _):
            do_chunk(2 * j, idxs0)
            do_chunk(2 * j + 1, idxs1)
            return 0

        lax.fori_loop(0, 25, body, 0)
        drain_prev()
        plsc.subcore_barrier()

        for off, n in ((0, 1024), (1024, 1024), (2048, 1024), (3072, 56)):
            stgo = rows.at[pl.ds(0, n), :]
            pltpu.sync_copy(acc_s.at[pl.ds(c0 + off, n)], stgo)
            pltpu.sync_copy(stgo, outq.at[pl.ds(c0 + off, n)])
        plsc.subcore_barrier()

    @pl.when(cid == 0)
    def _():
        one_pass(hv0, he0, out0)
        one_pass(hv1, he1, out1)

    @pl.when(cid == 1)
    def _():
        one_pass(hv2, he2, out2)
        one_pass(hv3, he3, out3)


@functools.partial(
    pl.kernel,
    out_type=[
        jax.ShapeDtypeStruct((320, 128), F32),
        jax.ShapeDtypeStruct((320, 128), F32),
        jax.ShapeDtypeStruct((PINR, 128), F32),
        jax.ShapeDtypeStruct((PINR, 128), F32),
    ],
    mesh=_mesh,
    compiler_params=pltpu.CompilerParams(use_tc_tiling_on_sc=False),
    scratch_types=[
        pltpu.VMEM_SHARED((UF,), F32),
        pltpu.VMEM_SHARED((SF,), F32),
        pltpu.VMEM((16, 128), I32),
        pltpu.VMEM((16, 128), I32),
        pltpu.VMEM((16, 128), I32),
        pltpu.VMEM((16, 128), I32),
        pltpu.VMEM((16, 128), I32),
        pltpu.VMEM((16, 128), I32),
        pltpu.VMEM((16, 128), F32),
        pltpu.VMEM((16, 128), F32),
        pltpu.VMEM((16, 128), F32),
        pltpu.VMEM((16, 128), F32),
        pltpu.VMEM((SF // 16,), F32),
        pltpu.SemaphoreType.DMA,
    ],
)
def _k4(uf, scf, nn0_2, nn1_2, pni2, pci2, nd0, na0, pd0, pa0,
        u_s, sc_s, ia, ib, f0, f1, f2, f3, g0, g1, g2, g3, stb, sem):
    cid, sid = _wid_axes()
    pltpu.sync_copy(uf.at[pl.ds(sid * (UF // 16), UF // 16)],
                    stb.at[pl.ds(0, UF // 16)])
    pltpu.sync_copy(stb.at[pl.ds(0, UF // 16)],
                    u_s.at[pl.ds(sid * (UF // 16), UF // 16)])
    pltpu.sync_copy(scf.at[pl.ds(sid * (SF // 16), SF // 16)], stb)
    pltpu.sync_copy(stb, sc_s.at[pl.ds(sid * (SF // 16), SF // 16)])
    plsc.subcore_barrier()

    @pl.when(cid == 0)
    def _():
        def nbody(k, _):
            c = sid + 16 * k

            @pl.when(c < 40)
            def _():
                r0 = c * 8
                pltpu.sync_copy(nn0_2.at[pl.ds(r0, 8)], ia.at[pl.ds(0, 8)])
                pltpu.sync_copy(nn1_2.at[pl.ds(r0, 8)], ib.at[pl.ds(0, 8)])

                @plsc.parallel_loop(0, 64, 1, unroll=8)
                def _(i):
                    r = i // 8
                    cc = (i % 8) * 16
                    va = ia[r, pl.ds(cc, 16)] * 8
                    vb = ib[r, pl.ds(cc, 16)] * 8
                    f0[r, pl.ds(cc, 16)] = va
                    f1[r, pl.ds(cc, 16)] = vb + 1
                    f2[r, pl.ds(cc, 16)] = va + 2
                    f3[r, pl.ds(cc, 16)] = vb + 3

                _fire_drain(
                    [pltpu.async_copy(u_s.at[f0.at[j]], g0.at[j], sem)
                     for j in range(8)] +
                    [pltpu.async_copy(u_s.at[f1.at[j]], g1.at[j], sem)
                     for j in range(8)] +
                    [pltpu.async_copy(u_s.at[f2.at[j]], g2.at[j], sem)
                     for j in range(8)] +
                    [pltpu.async_copy(u_s.at[f3.at[j]], g3.at[j], sem)
                     for j in range(8)])

                @plsc.parallel_loop(0, 64, 1, unroll=8)
                def _(i):
                    r = i // 8
                    cc = (i % 8) * 16
                    g0[r, pl.ds(cc, 16)] = (g0[r, pl.ds(cc, 16)] +
                                            g1[r, pl.ds(cc, 16)])
                    g2[r, pl.ds(cc, 16)] = (g2[r, pl.ds(cc, 16)] +
                                            g3[r, pl.ds(cc, 16)])

                pltpu.sync_copy(g0.at[pl.ds(0, 8)], nd0.at[pl.ds(r0, 8)])
                pltpu.sync_copy(g2.at[pl.ds(0, 8)], na0.at[pl.ds(r0, 8)])
            return 0

        lax.fori_loop(0, 3, nbody, 0)

    def pbody(k, _):
        ci = sid + 16 * k

        @pl.when(ci < 200)
        def _():
            c = 2 * ci + cid
            r0 = c * 16
            pltpu.sync_copy(pni2.at[pl.ds(r0, 16)], ia)
            pltpu.sync_copy(pci2.at[pl.ds(r0, 16)], ib)

            @plsc.parallel_loop(0, 128, 1, unroll=8)
            def _(i):
                r = i // 8
                cc = (i % 8) * 16
                va = ia[r, pl.ds(cc, 16)] * 8
                vb = ib[r, pl.ds(cc, 16)] * 2
                f0[r, pl.ds(cc, 16)] = va + 4
                f1[r, pl.ds(cc, 16)] = va + 5
                f2[r, pl.ds(cc, 16)] = vb
                f3[r, pl.ds(cc, 16)] = vb + 1

            _fire_drain(
                [pltpu.async_copy(u_s.at[f0.at[j]], g0.at[j], sem)
                 for j in range(16)] +
                [pltpu.async_copy(u_s.at[f1.at[j]], g1.at[j], sem)
                 for j in range(16)] +
                [pltpu.async_copy(sc_s.at[f2.at[j]], g2.at[j], sem)
                 for j in range(16)] +
                [pltpu.async_copy(sc_s.at[f3.at[j]], g3.at[j], sem)
                 for j in range(16)])

            @plsc.parallel_loop(0, 128, 1, unroll=8)
            def _(i):
                r = i // 8
                cc = (i % 8) * 16
                g0[r, pl.ds(cc, 16)] = (g0[r, pl.ds(cc, 16)] +
                                        g2[r, pl.ds(cc, 16)])
                g1[r, pl.ds(cc, 16)] = (g1[r, pl.ds(cc, 16)] +
                                        g3[r, pl.ds(cc, 16)])

            pltpu.sync_copy(g0, pd0.at[pl.ds(r0, 16)])
            pltpu.sync_copy(g1, pa0.at[pl.ds(r0, 16)])
        return 0

    lax.fori_loop(0, 13, pbody, 0)


def _t1a_body(x_ref, cnt_ref, w_ref, b_ref, lo_ref, hi_ref):
    h = jnp.tanh(jnp.dot(x_ref[...], w_ref[...],
                         preferred_element_type=F32) + b_ref[...])
    h = h * _rs(cnt_ref[...])
    lo_ref[...] = h[:, :32]
    hi_ref[...] = h[:, 32:]


def _t1b_body(x_ref, cnt3_ref, w_ref, b_ref, cw_ref, cb_ref,
              hs_ref, hd_ref, lo_ref):
    hn = jnp.tanh(jnp.dot(x_ref[...], w_ref[...],
                          preferred_element_type=F32) + b_ref[...])
    cnt3 = cnt3_ref[...]
    hs_ref[...] = hn * _rs(cnt3[:, 2:3])
    hd_ref[...] = hn * _rs(cnt3[:, 1:2])
    hv = jnp.dot(hn, cw_ref[...], preferred_element_type=F32) + cb_ref[...]
    lo_ref[...] = hv


def _t1c_body(x_ref, w1_ref, b1_ref, w2_ref, b2_ref, w3_ref, b3_ref, wsp_ref,
              q0_ref, q1_ref, q2_ref, q3_ref, sp_ref):
    hp = jnp.tanh(jnp.dot(x_ref[...], w1_ref[...],
                          preferred_element_type=F32) + b1_ref[...])
    t = _ssp(jnp.dot(hp, w2_ref[...], preferred_element_type=F32) + b2_ref[...])
    he = _ssp(jnp.dot(t, w3_ref[...], preferred_element_type=F32) + b3_ref[...])
    q0_ref[...] = he[:, :16]
    q1_ref[...] = he[:, 16:32]
    q2_ref[...] = he[:, 32:48]
    q3_ref[...] = he[:, 48:]
    sp_ref[...] = jnp.dot(hp, wsp_ref[...], preferred_element_type=F32)


def _t2a_body(a1l_ref, a1h_ref, af_ref, as_ref, cnt3_ref,
              wp_ref, bp_ref, wf_ref, bf_ref, ws_ref, bs_ref, w6_ref, b6_ref,
              u_ref):
    cnt3 = cnt3_ref[...]
    a1 = jnp.concatenate([a1l_ref[...], a1h_ref[...]], axis=-1)
    op = jnp.dot(a1 * _rs(cnt3[:, 0:1]), wp_ref[...],
                 preferred_element_type=F32) + bp_ref[...]
    of = jnp.dot(af_ref[...] * _rs(cnt3[:, 1:2]), wf_ref[...],
                 preferred_element_type=F32) + bf_ref[...]
    os_ = jnp.dot(as_ref[...] * _rs(cnt3[:, 2:3]), ws_ref[...],
                  preferred_element_type=F32) + bs_ref[...]
    h = jnp.maximum(jnp.maximum(op, of), os_)
    u_ref[...] = (jnp.dot(h, w6_ref[...], preferred_element_type=F32) +
                  b6_ref[...])


def _t2b_body(a40_ref, a41_ref, a42_ref, a43_ref, wo_ref, bo_ref, wsc_ref,
              sc_ref):
    a4 = jnp.concatenate([a40_ref[...], a41_ref[...], a42_ref[...],
                          a43_ref[...]], axis=-1)
    h = _ssp(jnp.dot(a4, wo_ref[...], preferred_element_type=F32) + bo_ref[...])
    sc_ref[...] = jnp.dot(h, wsc_ref[...], preferred_element_type=F32)


def _t3_body(pd_ref, pa_ref, sd_ref, sa_ref, dis_ref, ang_ref):
    dis_ref[...] = _softplus(pd_ref[...] + sd_ref[...])
    ang_ref[...] = pa_ref[...] + sa_ref[...]


def _t3b_body(nd_ref, dis_ref):
    dis_ref[...] = _softplus(nd_ref[...])


def _full(shape):
    nd = len(shape)
    return pl.BlockSpec(shape, lambda *_: (0,) * nd)


def kernel(cell_feat, net_feat, pin_feat, pin_cell_idx, pin_net_idx,
           father_src, father_dst, net_net_pair, params):
    p = params
    pci = pin_cell_idx.astype(I32)
    pni = pin_net_idx.astype(I32)
    fs = father_src.astype(I32)
    fd = father_dst.astype(I32)
    nn0 = net_net_pair[:, 0].astype(I32)
    nn1 = net_net_pair[:, 1].astype(I32)

    dead_n = 10048 + (jnp.arange(NNP - N_NN, dtype=I32) % 64)
    dead_f = 10048 + (jnp.arange(NNF - N_NN, dtype=I32) % 64)
    dead_np = 10048 + (jnp.arange(NPP - N_PIN, dtype=I32) % 64)
    dead_cp = 50000 + (jnp.arange(NPP - N_PIN, dtype=I32) % 48)
    pci2 = jnp.concatenate([pci, dead_cp]).reshape(PINR, 128)
    pni2 = jnp.concatenate([pni, dead_np]).reshape(PINR, 128)
    fs2 = jnp.concatenate([fs, dead_f]).reshape(384, 128)
    fd2 = jnp.concatenate([fd, dead_f]).reshape(384, 128)
    nn0_2 = jnp.concatenate([nn0, dead_n]).reshape(320, 128)
    nn1_2 = jnp.concatenate([nn1, dead_n]).reshape(320, 128)

    ones_h = jnp.ones((16, 128), F32)
    z1 = jnp.zeros((CSL,), F32)
    z2d = jnp.zeros((CSL, 32), F32)
    z16 = jnp.zeros((CSL, 16), F32)
    z64 = jnp.zeros((NSL, 64), F32)

    cell_p = jnp.pad(cell_feat, ((0, CP - N_CELL), (0, 0)))
    net_p = jnp.pad(net_feat, ((0, NPAD - N_NET), (0, 0)))
    pin_p = jnp.pad(pin_feat, ((0, NPP - N_PIN), (0, 0)))

    wd, wa = p["net_dis"]["W"][:, 0], p["net_angle"]["W"][:, 0]
    wpd, wpa = p["pin_dis"]["W"][:, 0], p["pin_angle"]["W"][:, 0]
    zc = jnp.zeros((64,), F32)
    w6 = jnp.stack([wd[:64], wd[64:], wa[:64], wa[64:], wpd[:64], wpa[:64],
                    zc, zc], axis=-1)
    e = jnp.eye(8, dtype=F32)
    b6 = (e[0] * p["net_dis"]["b"][0] + e[2] * p["net_angle"]["b"][0] +
          e[4] * p["pin_dis"]["b"][0] + e[5] * p["pin_angle"]["b"][0])
    b6 = b6.reshape(1, 8)
    wsp = jnp.stack([wpd[64:80], wpa[64:80]], axis=-1)
    wsc = jnp.stack([wpd[80:], wpa[80:]], axis=-1)

    cc, cn, cf, cd = _k0(pci2, pni2, fs2, fd2, ones_h, z1)
    cnt3 = jnp.stack([cn, cd, cf], axis=-1)

    hc_lo, hc_hi = pl.pallas_call(
        _t1a_body,
        grid=(23,),
        in_specs=[
            pl.BlockSpec((2176, 16), lambda i: (i, 0)),
            pl.BlockSpec((2176, 1), lambda i: (i, 0)),
            _full((16, 64)),
            _full((1, 64)),
        ],
        out_specs=[
            pl.BlockSpec((2176, 32), lambda i: (i, 0)),
            pl.BlockSpec((2176, 32), lambda i: (i, 0)),
        ],
        out_shape=[
            jax.ShapeDtypeStruct((CP, 32), F32),
            jax.ShapeDtypeStruct((CP, 32), F32),
        ],
    )(cell_p, cc.reshape(CP, 1), p["cell_lin"]["W"],
      p["cell_lin"]["b"].reshape(1, 64))

    hn_src, hn_dst, hv = pl.pallas_call(
        _t1b_body,
        grid=(8,),
        in_specs=[
            pl.BlockSpec((1264, 8), lambda i: (i, 0)),
            pl.BlockSpec((1264, 3), lambda i: (i, 0)),
            _full((8, 64)),
            _full((1, 64)),
            _full((64, 64)),
            _full((1, 64)),
        ],
        out_specs=[
            pl.BlockSpec((1264, 64), lambda i: (i, 0)),
            pl.BlockSpec((1264, 64), lambda i: (i, 0)),
            pl.BlockSpec((1264, 64), lambda i: (i, 0)),
        ],
        out_shape=[
            jax.ShapeDtypeStruct((NPAD, 64), F32),
            jax.ShapeDtypeStruct((NPAD, 64), F32),
            jax.ShapeDtypeStruct((NPAD, 64), F32),
        ],
    )(net_p, cnt3, p["net_lin"]["W"], p["net_lin"]["b"].reshape(1, 64),
      p["cf_node"]["W"], p["cf_node"]["b"].reshape(1, 64))

    he0, he1, he2, he3, s_pin = pl.pallas_call(
        _t1c_body,
        grid=(400,),
        in_specs=[
            pl.BlockSpec((2048, 8), lambda i: (i, 0)),
            _full((8, 16)),
            _full((1, 16)),
            _full((16, 64)),
            _full((1, 64)),
            _full((64, 64)),
            _full((1, 64)),
            _full((16, 2)),
        ],
        out_specs=[
            pl.BlockSpec((2048, 16), lambda i: (i, 0)),
            pl.BlockSpec((2048, 16), lambda i: (i, 0)),
            pl.BlockSpec((2048, 16), lambda i: (i, 0)),
            pl.BlockSpec((2048, 16), lambda i: (i, 0)),
            pl.BlockSpec((2048, 2), lambda i: (i, 0)),
        ],
        out_shape=[
            jax.ShapeDtypeStruct((NPP, 16), F32),
            jax.ShapeDtypeStruct((NPP, 16), F32),
            jax.ShapeDtypeStruct((NPP, 16), F32),
            jax.ShapeDtypeStruct((NPP, 16), F32),
            jax.ShapeDtypeStruct((NPP, 2), F32),
        ],
    )(pin_p, p["pin_lin"]["W"], p["pin_lin"]["b"].reshape(1, 16),
      p["cf_edge1"]["W"], p["cf_edge1"]["b"].reshape(1, 64),
      p["cf_edge2"]["W"], p["cf_edge2"]["b"].reshape(1, 64), wsp)

    fsd2 = jnp.concatenate([fs2, fd2], axis=0)
    hn2 = jnp.concatenate([hn_src, hn_dst], axis=0)
    out2 = _k1(hn2, fsd2, z64)[0]
    accf, accs = out2[:NPAD], out2[NPAD:]
    acc1_lo, acc1_hi = _k2(hc_lo, hc_hi, pci2, pni2, z2d)
    hv0, hv1, hv2, hv3 = (hv[:, :16], hv[:, 16:32], hv[:, 32:48], hv[:, 48:])
    a40, a41, a42, a43 = _k3(hv0, hv1, hv2, hv3, he0, he1, he2, he3,
                             pci2, pni2, z16)

    u = pl.pallas_call(
        _t2a_body,
        grid=(4,),
        in_specs=[
            pl.BlockSpec((2528, 32), lambda i: (i, 0)),
            pl.BlockSpec((2528, 32), lambda i: (i, 0)),
            pl.BlockSpec((2528, 64), lambda i: (i, 0)),
            pl.BlockSpec((2528, 64), lambda i: (i, 0)),
            pl.BlockSpec((2528, 3), lambda i: (i, 0)),
            _full((64, 64)), _full((1, 64)),
            _full((64, 64)), _full((1, 64)),
            _full((64, 64)), _full((1, 64)),
            _full((64, 8)), _full((1, 8)),
        ],
        out_specs=[pl.BlockSpec((2528, 8), lambda i: (i, 0))],
        out_shape=[jax.ShapeDtypeStruct((NPAD, 8), F32)],
    )(acc1_lo, acc1_hi, accf, accs, cnt3,
      p["gc_pins"]["W"], p["gc_pins"]["b"].reshape(1, 64),
      p["gc_father"]["W"], p["gc_father"]["b"].reshape(1, 64),
      p["gc_son"]["W"], p["gc_son"]["b"].reshape(1, 64),
      w6, b6)[0]

    s_cell = pl.pallas_call(
        _t2b_body,
        grid=(23,),
        in_specs=[
            pl.BlockSpec((2176, 16), lambda i: (i, 0)),
            pl.BlockSpec((2176, 16), lambda i: (i, 0)),
            pl.BlockSpec((2176, 16), lambda i: (i, 0)),
            pl.BlockSpec((2176, 16), lambda i: (i, 0)),
            _full((64, 64)), _full((1, 64)), _full((64, 2)),
        ],
        out_specs=[pl.BlockSpec((2176, 2), lambda i: (i, 0))],
        out_shape=[jax.ShapeDtypeStruct((CP, 2), F32)],
    )(a40, a41, a42, a43, p["cf_out"]["W"], p["cf_out"]["b"].reshape(1, 64),
      wsc)[0]

    nd0, na0, pd0, pa0 = _k4(u.reshape(UF), s_cell.reshape(SF),
                             nn0_2, nn1_2, pni2, pci2)

    net_dis2 = pl.pallas_call(
        _t3b_body,
        in_specs=[_full((320, 128))],
        out_specs=[_full((320, 128))],
        out_shape=[jax.ShapeDtypeStruct((320, 128), F32)],
    )(nd0)[0]

    sd = s_pin[:, 0].reshape(PINR, 128)
    sa = s_pin[:, 1].reshape(PINR, 128)
    dis2, ang2 = pl.pallas_call(
        _t3_body,
        grid=(8,),
        in_specs=[pl.BlockSpec((800, 128), lambda i: (i, 0))] * 4,
        out_specs=[pl.BlockSpec((800, 128), lambda i: (i, 0))] * 2,
        out_shape=[
            jax.ShapeDtypeStruct((PINR, 128), F32),
            jax.ShapeDtypeStruct((PINR, 128), F32),
        ],
    )(pd0, pa0, sd, sa)

    return (net_dis2.reshape(-1)[:N_NN], na0.reshape(-1)[:N_NN],
            dis2.reshape(-1)[:N_PIN], ang2.reshape(-1)[:N_PIN])


# speedup vs baseline: 3.7318x; 1.0243x over previous
import functools

import jax
import jax.numpy as jnp
from jax import lax
from jax.experimental import pallas as pl
from jax.experimental.pallas import tpu as pltpu
from jax.experimental.pallas import tpu_sc as plsc

F32 = jnp.float32
I32 = jnp.int32
LOG2 = 0.6931471805599453

N_CELL, N_NET, N_PIN, N_NN = 50000, 10000, 800000, 40000
CP, NPAD = 50048, 10112
CSL, NSL = 3128, 632
NNP = 40960
NNF = 49152
NPP = 819200
PINR = NPP // 128
UF = NPAD * 8
SF = CP * 2

_mesh = plsc.VectorSubcoreMesh(core_axis_name="c", subcore_axis_name="s")


def _softplus(x):
    return jnp.log(1.0 + jnp.exp(jnp.minimum(x, 30.0)))


def _ssp(x):
    return _softplus(x) - LOG2


def _rs(c):
    return lax.rsqrt(jnp.maximum(c, 1.0))


def _wid_axes():
    return lax.axis_index("c"), lax.axis_index("s")


def _fire_drain(descs):
    for d in descs:
        d.wait()


@functools.partial(
    pl.kernel,
    out_type=[
        jax.ShapeDtypeStruct((CP,), F32),
        jax.ShapeDtypeStruct((NPAD,), F32),
        jax.ShapeDtypeStruct((NPAD,), F32),
        jax.ShapeDtypeStruct((NPAD,), F32),
    ],
    mesh=_mesh,
    compiler_params=pltpu.CompilerParams(use_tc_tiling_on_sc=False),
    scratch_types=[
        pltpu.VMEM_SHARED((CP,), F32),
        pltpu.VMEM_SHARED((NPAD,), F32),
        pltpu.VMEM_SHARED((NPAD,), F32),
        pltpu.VMEM_SHARED((NPAD,), F32),
        pltpu.VMEM((16, 128), I32),
        pltpu.VMEM((16, 128), F32),
        pltpu.VMEM((CSL,), F32),
        pltpu.SemaphoreType.DMA,
    ],
)
def _k0(pci2, pni2, fs2, fd2, ones_h, z1, out_cc, out_cn, out_cf, out_cd,
        hc_s, hn_s, hf_s, hd_s, idx_v, ones_v, zb, sem):
    cid, sid = _wid_axes()
    pltpu.sync_copy(ones_h, ones_v)
    pltpu.sync_copy(z1, zb)
    pltpu.sync_copy(zb, hc_s.at[pl.ds(sid * CSL, CSL)])
    pltpu.sync_copy(zb.at[pl.ds(0, NSL)], hn_s.at[pl.ds(sid * NSL, NSL)])
    pltpu.sync_copy(zb.at[pl.ds(0, NSL)], hf_s.at[pl.ds(sid * NSL, NSL)])
    pltpu.sync_copy(zb.at[pl.ds(0, NSL)], hd_s.at[pl.ds(sid * NSL, NSL)])
    plsc.subcore_barrier()

    def pin_hist(src2d, hist):
        def body(k, _):
            c = sid + 16 * k
            pltpu.sync_copy(src2d.at[pl.ds(c * 16, 16)], idx_v)
            _fire_drain([
                pltpu.async_copy(ones_v.at[j], hist.at[idx_v.at[j]], sem,
                                 add=True)
                for j in range(16)
            ])
            return 0

        lax.fori_loop(0, 25, body, 0)

    @pl.when(cid == 0)
    def _():
        pin_hist(pci2, hc_s)

    @pl.when(cid == 1)
    def _():
        pin_hist(pni2, hn_s)

        def body2(k, _):
            c = sid + 16 * k

            @pl.when(c < 48)
            def _():
                r0 = c * 8
                pltpu.sync_copy(fs2.at[pl.ds(r0, 8)], idx_v.at[pl.ds(0, 8)])
                _fire_drain([
                    pltpu.async_copy(ones_v.at[j], hf_s.at[idx_v.at[j]], sem,
                                     add=True)
                    for j in range(8)
                ])
                pltpu.sync_copy(fd2.at[pl.ds(r0, 8)], idx_v.at[pl.ds(0, 8)])
                _fire_drain([
                    pltpu.async_copy(ones_v.at[j], hd_s.at[idx_v.at[j]], sem,
                                     add=True)
                    for j in range(8)
                ])
            return 0

        lax.fori_loop(0, 3, body2, 0)

    plsc.subcore_barrier()

    @pl.when(cid == 0)
    def _():
        pltpu.sync_copy(hc_s.at[pl.ds(sid * CSL, CSL)], zb)
        pltpu.sync_copy(zb, out_cc.at[pl.ds(sid * CSL, CSL)])

    @pl.when(cid == 1)
    def _():
        s0 = sid * NSL
        pltpu.sync_copy(hn_s.at[pl.ds(s0, NSL)], zb.at[pl.ds(0, NSL)])
        pltpu.sync_copy(zb.at[pl.ds(0, NSL)], out_cn.at[pl.ds(s0, NSL)])
        pltpu.sync_copy(hf_s.at[pl.ds(s0, NSL)], zb.at[pl.ds(0, NSL)])
        pltpu.sync_copy(zb.at[pl.ds(0, NSL)], out_cf.at[pl.ds(s0, NSL)])
        pltpu.sync_copy(hd_s.at[pl.ds(s0, NSL)], zb.at[pl.ds(0, NSL)])
        pltpu.sync_copy(zb.at[pl.ds(0, NSL)], out_cd.at[pl.ds(s0, NSL)])


@functools.partial(
    pl.kernel,
    out_type=[jax.ShapeDtypeStruct((2 * NPAD, 64), F32)],
    mesh=_mesh,
    compiler_params=pltpu.CompilerParams(use_tc_tiling_on_sc=False),
    scratch_types=[
        pltpu.VMEM_SHARED((NPAD, 64), F32),
        pltpu.VMEM((8, 128), I32),
        pltpu.VMEM((8, 128), I32),
        pltpu.VMEM((1024, 64), F32),
        pltpu.SemaphoreType.DMA,
        pltpu.SemaphoreType.DMA,
    ],
)
def _k1(hn2, fsd2, z64, out2,
        acc_s, idxg, idxs, rows, semg, sems):
    cid, sid = _wid_axes()
    s0 = sid * NSL

    stg = rows.at[pl.ds(0, NSL), :]
    pltpu.sync_copy(z64, stg)
    pltpu.sync_copy(stg, acc_s.at[pl.ds(s0, NSL)])
    plsc.subcore_barrier()

    def body(k, _):
        c = sid + 16 * k
        r0 = c * 8
        pltpu.sync_copy(fsd2.at[pl.ds(cid * 384 + r0, 8)], idxg)
        pltpu.sync_copy(fsd2.at[pl.ds((1 - cid) * 384 + r0, 8)], idxs)

        off = cid * NPAD

        @plsc.parallel_loop(0, 64, 1, unroll=8)
        def _(i):
            r = i // 8
            cc = (i % 8) * 16
            idxg[r, pl.ds(cc, 16)] = idxg[r, pl.ds(cc, 16)] + off

        _fire_drain([
            pltpu.async_copy(hn2.at[idxg.at[j]],
                             rows.at[pl.ds(j * 128, 128)], semg)
            for j in range(8)
        ])
        _fire_drain([
            pltpu.async_copy(rows.at[pl.ds(j * 128, 128)],
                             acc_s.at[idxs.at[j]], sems, add=True)
            for j in range(8)
        ])
        return 0

    lax.fori_loop(0, 3, body, 0)
    plsc.subcore_barrier()

    pltpu.sync_copy(acc_s.at[pl.ds(s0, NSL)], stg)
    pltpu.sync_copy(stg, out2.at[pl.ds(cid * NPAD + s0, NSL)])


@functools.partial(
    pl.kernel,
    out_type=[
        jax.ShapeDtypeStruct((NPAD, 32), F32),
        jax.ShapeDtypeStruct((NPAD, 32), F32),
    ],
    mesh=_mesh,
    compiler_params=pltpu.CompilerParams(use_tc_tiling_on_sc=False),
    scratch_types=[
        pltpu.VMEM_SHARED((NPAD, 32), F32),
        pltpu.VMEM((16, 128), I32),
        pltpu.VMEM((16, 128), I32),
        pltpu.VMEM((2048, 32), F32),
        pltpu.SemaphoreType.DMA,
        pltpu.SemaphoreType.DMA,
    ],
)
def _k2(hc_lo, hc_hi, pci2, pni2, z2d, out_lo, out_hi,
        acc_s, idxg, idxs, rows, semg, sems):
    cid, sid = _wid_axes()
    s0 = sid * NSL

    stgz = rows.at[pl.ds(0, NSL), :]
    pltpu.sync_copy(z2d.at[pl.ds(0, NSL)], stgz)
    pltpu.sync_copy(stgz, acc_s.at[pl.ds(s0, NSL)])
    plsc.subcore_barrier()

    def body(k, _):
        c = sid + 16 * k
        r0 = c * 16
        pltpu.sync_copy(pci2.at[pl.ds(r0, 16)], idxg)
        pltpu.sync_copy(pni2.at[pl.ds(r0, 16)], idxs)

        @pl.when(cid == 0)
        def _():
            _fire_drain([
                pltpu.async_copy(hc_lo.at[idxg.at[j]],
                                 rows.at[pl.ds(j * 128, 128)], semg)
                for j in range(16)
            ])

        @pl.when(cid == 1)
        def _():
            _fire_drain([
                pltpu.async_copy(hc_hi.at[idxg.at[j]],
                                 rows.at[pl.ds(j * 128, 128)], semg)
                for j in range(16)
            ])

        _fire_drain([
            pltpu.async_copy(rows.at[pl.ds(j * 128, 128)],
                             acc_s.at[idxs.at[j]], sems, add=True)
            for j in range(16)
        ])
        return 0

    lax.fori_loop(0, 25, body, 0)
    plsc.subcore_barrier()

    stgo = rows.at[pl.ds(0, NSL), :]
    pltpu.sync_copy(acc_s.at[pl.ds(s0, NSL)], stgo)

    @pl.when(cid == 0)
    def _():
        pltpu.sync_copy(stgo, out_lo.at[pl.ds(s0, NSL)])

    @pl.when(cid == 1)
    def _():
        pltpu.sync_copy(stgo, out_hi.at[pl.ds(s0, NSL)])


@functools.partial(
    pl.kernel,
    out_type=[jax.ShapeDtypeStruct((CP, 16), F32) for _ in range(4)],
    mesh=_mesh,
    compiler_params=pltpu.CompilerParams(use_tc_tiling_on_sc=False),
    scratch_types=[
        pltpu.VMEM_SHARED((CP, 16), F32),
        pltpu.VMEM_SHARED((NPAD, 16), F32),
        pltpu.VMEM((8, 128), I32),
        pltpu.VMEM((8, 128), I32),
        pltpu.VMEM((8, 128), I32),
        pltpu.VMEM((1024, 16), F32),
        pltpu.VMEM((1024, 16), F32),
        pltpu.SemaphoreType.DMA,
        pltpu.SemaphoreType.DMA,
        pltpu.SemaphoreType.DMA,
    ],
)
def _k3(hv0, hv1, hv2, hv3, he0, he1, he2, he3, pci2, pni2, z16,
        out0, out1, out2, out3,
        acc_s, tab_s, idxg, idxs0, idxs1, rows, heb, semg, sems, semh):
    cid, sid = _wid_axes()
    c0 = sid * CSL
    s0 = sid * NSL

    def one_pass(hvq, heq, outq):
        stg = rows.at[pl.ds(0, NSL), :]
        pltpu.sync_copy(hvq.at[pl.ds(s0, NSL)], stg)
        pltpu.sync_copy(stg, tab_s.at[pl.ds(s0, NSL)])
        for off, n in ((0, 1024), (1024, 1024), (2048, 1024), (3072, 56)):
            stgz = rows.at[pl.ds(0, n), :]
            pltpu.sync_copy(z16.at[pl.ds(off, n)], stgz)
            pltpu.sync_copy(stgz, acc_s.at[pl.ds(c0 + off, n)])
        plsc.subcore_barrier()

        def drain_prev():
            pltpu.make_async_copy(heq.at[pl.ds(0, 1024), :], heb,
                                  sems).wait()

        pltpu.sync_copy(pci2.at[pl.ds(0, 8)], idxs1)
        for j in range(8):
            pltpu.async_copy(rows.at[pl.ds(j * 128, 128), :],
                             acc_s.at[idxs1.at[j]], sems, add=True)

        def do_chunk(k, idxs):
            c = sid + 16 * k
            r0 = c * 8
            pltpu.sync_copy(pni2.at[pl.ds(r0, 8)], idxg)
            pltpu.sync_copy(pci2.at[pl.ds(r0, 8)], idxs)
            hed = pltpu.async_copy(heq.at[pl.ds(c * 1024, 1024), :], heb,
                                   semh)
            drain_prev()
            _fire_drain([
                pltpu.async_copy(tab_s.at[idxg.at[j]],
                                 rows.at[pl.ds(j * 128, 128)], semg)
                for j in range(8)
            ])
            hed.wait()

            @plsc.parallel_loop(0, 1024, 1, unroll=8)
            def _(q):
                rows[q, pl.ds(0, 16)] = (heb[q, pl.ds(0, 16)] *
                                         rows[q, pl.ds(0, 16)])

            for j in range(8):
                pltpu.async_copy(rows.at[pl.ds(j * 128, 128), :],
                                 acc_s.at[idxs.at[j]], sems, add=True)

        def body(j, _):
            do_chunk(2 * j, idxs0)
            do_chunk(2 * j + 1, idxs1)
            return 0

        lax.fori_loop(0, 25, body, 0)
        drain_prev()
        plsc.subcore_barrier()

        for off, n in ((0, 1024), (1024, 1024), (2048, 1024), (3072, 56)):
            stgo = rows.at[pl.ds(0, n), :]
            pltpu.sync_copy(acc_s.at[pl.ds(c0 + off, n)], stgo)
            pltpu.sync_copy(stgo, outq.at[pl.ds(c0 + off, n)])
        plsc.subcore_barrier()

    @pl.when(cid == 0)
    def _():
        one_pass(hv0, he0, out0)
        one_pass(hv1, he1, out1)

    @pl.when(cid == 1)
    def _():
        one_pass(hv2, he2, out2)
        one_pass(hv3, he3, out3)


@functools.partial(
    pl.kernel,
    out_type=[
        jax.ShapeDtypeStruct((320, 128), F32),
        jax.ShapeDtypeStruct((320, 128), F32),
        jax.ShapeDtypeStruct((PINR, 128), F32),
        jax.ShapeDtypeStruct((PINR, 128), F32),
    ],
    mesh=_mesh,
    compiler_params=pltpu.CompilerParams(use_tc_tiling_on_sc=False),
    scratch_types=[
        pltpu.VMEM_SHARED((UF,), F32),
        pltpu.VMEM_SHARED((SF,), F32),
        pltpu.VMEM((16, 128), I32),
        pltpu.VMEM((16, 128), I32),
        pltpu.VMEM((16, 128), I32),
        pltpu.VMEM((16, 128), I32),
        pltpu.VMEM((16, 128), I32),
        pltpu.VMEM((16, 128), I32),
        pltpu.VMEM((16, 128), F32),
        pltpu.VMEM((16, 128), F32),
        pltpu.VMEM((16, 128), F32),
        pltpu.VMEM((16, 128), F32),
        pltpu.VMEM((SF // 16,), F32),
        pltpu.SemaphoreType.DMA,
    ],
)
def _k4(uf, scf, nn0_2, nn1_2, pni2, pci2, nd0, na0, pd0, pa0,
        u_s, sc_s, ia, ib, f0, f1, f2, f3, g0, g1, g2, g3, stb, sem):
    cid, sid = _wid_axes()
    pltpu.sync_copy(uf.at[pl.ds(sid * (UF // 16), UF // 16)],
                    stb.at[pl.ds(0, UF // 16)])
    pltpu.sync_copy(stb.at[pl.ds(0, UF // 16)],
                    u_s.at[pl.ds(sid * (UF // 16), UF // 16)])
    pltpu.sync_copy(scf.at[pl.ds(sid * (SF // 16), SF // 16)], stb)
    pltpu.sync_copy(stb, sc_s.at[pl.ds(sid * (SF // 16), SF // 16)])
    plsc.subcore_barrier()

    @pl.when(cid == 0)
    def _():
        def nbody(k, _):
            c = sid + 16 * k

            @pl.when(c < 40)
            def _():
                r0 = c * 8
                pltpu.sync_copy(nn0_2.at[pl.ds(r0, 8)], ia.at[pl.ds(0, 8)])
                pltpu.sync_copy(nn1_2.at[pl.ds(r0, 8)], ib.at[pl.ds(0, 8)])

                @plsc.parallel_loop(0, 64, 1, unroll=8)
                def _(i):
                    r = i // 8
                    cc = (i % 8) * 16
                    va = ia[r, pl.ds(cc, 16)] * 8
                    vb = ib[r, pl.ds(cc, 16)] * 8
                    f0[r, pl.ds(cc, 16)] = va
                    f1[r, pl.ds(cc, 16)] = vb + 1
                    f2[r, pl.ds(cc, 16)] = va + 2
                    f3[r, pl.ds(cc, 16)] = vb + 3

                _fire_drain(
                    [pltpu.async_copy(u_s.at[f0.at[j]], g0.at[j], sem)
                     for j in range(8)] +
                    [pltpu.async_copy(u_s.at[f1.at[j]], g1.at[j], sem)
                     for j in range(8)] +
                    [pltpu.async_copy(u_s.at[f2.at[j]], g2.at[j], sem)
                     for j in range(8)] +
                    [pltpu.async_copy(u_s.at[f3.at[j]], g3.at[j], sem)
                     for j in range(8)])

                @plsc.parallel_loop(0, 64, 1, unroll=8)
                def _(i):
                    r = i // 8
                    cc = (i % 8) * 16
                    g0[r, pl.ds(cc, 16)] = (g0[r, pl.ds(cc, 16)] +
                                            g1[r, pl.ds(cc, 16)])
                    g2[r, pl.ds(cc, 16)] = (g2[r, pl.ds(cc, 16)] +
                                            g3[r, pl.ds(cc, 16)])

                pltpu.sync_copy(g0.at[pl.ds(0, 8)], nd0.at[pl.ds(r0, 8)])
                pltpu.sync_copy(g2.at[pl.ds(0, 8)], na0.at[pl.ds(r0, 8)])
            return 0

        lax.fori_loop(0, 3, nbody, 0)

    def pbody(k, _):
        ci = sid + 16 * k

        @pl.when(ci < 200)
        def _():
            c = 2 * ci + cid
            r0 = c * 16
            pltpu.sync_copy(pni2.at[pl.ds(r0, 16)], ia)
            pltpu.sync_copy(pci2.at[pl.ds(r0, 16)], ib)

            @plsc.parallel_loop(0, 128, 1, unroll=8)
            def _(i):
                r = i // 8
                cc = (i % 8) * 16
                va = ia[r, pl.ds(cc, 16)] * 8
                vb = ib[r, pl.ds(cc, 16)] * 2
                f0[r, pl.ds(cc, 16)] = va + 4
                f1[r, pl.ds(cc, 16)] = va + 5
                f2[r, pl.ds(cc, 16)] = vb
                f3[r, pl.ds(cc, 16)] = vb + 1

            _fire_drain(
                [pltpu.async_copy(u_s.at[f0.at[j]], g0.at[j], sem)
                 for j in range(16)] +
                [pltpu.async_copy(u_s.at[f1.at[j]], g1.at[j], sem)
                 for j in range(16)] +
                [pltpu.async_copy(sc_s.at[f2.at[j]], g2.at[j], sem)
                 for j in range(16)] +
                [pltpu.async_copy(sc_s.at[f3.at[j]], g3.at[j], sem)
                 for j in range(16)])

            @plsc.parallel_loop(0, 128, 1, unroll=8)
            def _(i):
                r = i // 8
                cc = (i % 8) * 16
                g0[r, pl.ds(cc, 16)] = (g0[r, pl.ds(cc, 16)] +
                                        g2[r, pl.ds(cc, 16)])
                g1[r, pl.ds(cc, 16)] = (g1[r, pl.ds(cc, 16)] +
                                        g3[r, pl.ds(cc, 16)])

            pltpu.sync_copy(g0, pd0.at[pl.ds(r0, 16)])
            pltpu.sync_copy(g1, pa0.at[pl.ds(r0, 16)])
        return 0

    lax.fori_loop(0, 13, pbody, 0)


def _t1a_body(x_ref, cnt_ref, w_ref, b_ref, lo_ref, hi_ref):
    h = jnp.tanh(jnp.dot(x_ref[...], w_ref[...],
                         preferred_element_type=F32) + b_ref[...])
    h = h * _rs(cnt_ref[...])
    lo_ref[...] = h[:, :32]
    hi_ref[...] = h[:, 32:]


def _t1b_body(x_ref, cnt3_ref, w_ref, b_ref, cw_ref, cb_ref,
              hs_ref, hd_ref, lo_ref):
    hn = jnp.tanh(jnp.dot(x_ref[...], w_ref[...],
                          preferred_element_type=F32) + b_ref[...])
    cnt3 = cnt3_ref[...]
    hs_ref[...] = hn * _rs(cnt3[:, 2:3])
    hd_ref[...] = hn * _rs(cnt3[:, 1:2])
    hv = jnp.dot(hn, cw_ref[...], preferred_element_type=F32) + cb_ref[...]
    lo_ref[...] = hv


def _t1c_body(x_ref, w1_ref, b1_ref, w2_ref, b2_ref, w3_ref, b3_ref, wsp_ref,
              q0_ref, q1_ref, q2_ref, q3_ref, sp_ref):
    hp = jnp.tanh(jnp.dot(x_ref[...], w1_ref[...],
                          preferred_element_type=F32) + b1_ref[...])
    t = _ssp(jnp.dot(hp, w2_ref[...], preferred_element_type=F32) + b2_ref[...])
    he = _ssp(jnp.dot(t, w3_ref[...], preferred_element_type=F32) + b3_ref[...])
    q0_ref[...] = he[:, :16]
    q1_ref[...] = he[:, 16:32]
    q2_ref[...] = he[:, 32:48]
    q3_ref[...] = he[:, 48:]
    sp_ref[...] = jnp.dot(hp, wsp_ref[...], preferred_element_type=F32)


def _t2a_body(a1l_ref, a1h_ref, af_ref, as_ref, cnt3_ref,
              wp_ref, bp_ref, wf_ref, bf_ref, ws_ref, bs_ref, w6_ref, b6_ref,
              u_ref):
    cnt3 = cnt3_ref[...]
    a1 = jnp.concatenate([a1l_ref[...], a1h_ref[...]], axis=-1)
    op = jnp.dot(a1 * _rs(cnt3[:, 0:1]), wp_ref[...],
                 preferred_element_type=F32) + bp_ref[...]
    of = jnp.dot(af_ref[...] * _rs(cnt3[:, 1:2]), wf_ref[...],
                 preferred_element_type=F32) + bf_ref[...]
    os_ = jnp.dot(as_ref[...] * _rs(cnt3[:, 2:3]), ws_ref[...],
                  preferred_element_type=F32) + bs_ref[...]
    h = jnp.maximum(jnp.maximum(op, of), os_)
    u_ref[...] = (jnp.dot(h, w6_ref[...], preferred_element_type=F32) +
                  b6_ref[...])


def _t2b_body(a40_ref, a41_ref, a42_ref, a43_ref, wo_ref, bo_ref, wsc_ref,
              sc_ref):
    a4 = jnp.concatenate([a40_ref[...], a41_ref[...], a42_ref[...],
                          a43_ref[...]], axis=-1)
    h = _ssp(jnp.dot(a4, wo_ref[...], preferred_element_type=F32) + bo_ref[...])
    sc_ref[...] = jnp.dot(h, wsc_ref[...], preferred_element_type=F32)


def _t3_body(pd_ref, pa_ref, sd_ref, sa_ref, dis_ref, ang_ref):
    dis_ref[...] = _softplus(pd_ref[...] + sd_ref[...])
    ang_ref[...] = pa_ref[...] + sa_ref[...]


def _t3b_body(nd_ref, dis_ref):
    dis_ref[...] = _softplus(nd_ref[...])


def _full(shape):
    nd = len(shape)
    return pl.BlockSpec(shape, lambda *_: (0,) * nd)


def kernel(cell_feat, net_feat, pin_feat, pin_cell_idx, pin_net_idx,
           father_src, father_dst, net_net_pair, params):
    p = params
    pci = pin_cell_idx.astype(I32)
    pni = pin_net_idx.astype(I32)
    fs = father_src.astype(I32)
    fd = father_dst.astype(I32)
    nn0 = net_net_pair[:, 0].astype(I32)
    nn1 = net_net_pair[:, 1].astype(I32)

    dead_n = 10048 + (jnp.arange(NNP - N_NN, dtype=I32) % 64)
    dead_f = 10048 + (jnp.arange(NNF - N_NN, dtype=I32) % 64)
    dead_np = 10048 + (jnp.arange(NPP - N_PIN, dtype=I32) % 64)
    dead_cp = 50000 + (jnp.arange(NPP - N_PIN, dtype=I32) % 48)
    pci2 = jnp.concatenate([pci, dead_cp]).reshape(PINR, 128)
    pni2 = jnp.concatenate([pni, dead_np]).reshape(PINR, 128)
    fs2 = jnp.concatenate([fs, dead_f]).reshape(384, 128)
    fd2 = jnp.concatenate([fd, dead_f]).reshape(384, 128)
    nn0_2 = jnp.concatenate([nn0, dead_n]).reshape(320, 128)
    nn1_2 = jnp.concatenate([nn1, dead_n]).reshape(320, 128)

    ones_h = jnp.ones((16, 128), F32)
    z1 = jnp.zeros((CSL,), F32)
    z2d = jnp.zeros((CSL, 32), F32)
    z16 = jnp.zeros((CSL, 16), F32)
    z64 = jnp.zeros((NSL, 64), F32)

    cell_p = jnp.pad(cell_feat, ((0, CP - N_CELL), (0, 0)))
    net_p = jnp.pad(net_feat, ((0, NPAD - N_NET), (0, 0)))
    pin_p = jnp.pad(pin_feat, ((0, NPP - N_PIN), (0, 0)))

    wd, wa = p["net_dis"]["W"][:, 0], p["net_angle"]["W"][:, 0]
    wpd, wpa = p["pin_dis"]["W"][:, 0], p["pin_angle"]["W"][:, 0]
    zc = jnp.zeros((64,), F32)
    w6 = jnp.stack([wd[:64], wd[64:], wa[:64], wa[64:], wpd[:64], wpa[:64],
                    zc, zc], axis=-1)
    e = jnp.eye(8, dtype=F32)
    b6 = (e[0] * p["net_dis"]["b"][0] + e[2] * p["net_angle"]["b"][0] +
          e[4] * p["pin_dis"]["b"][0] + e[5] * p["pin_angle"]["b"][0])
    b6 = b6.reshape(1, 8)
    wsp = jnp.stack([wpd[64:80], wpa[64:80]], axis=-1)
    wsc = jnp.stack([wpd[80:], wpa[80:]], axis=-1)

    cc, cn, cf, cd = _k0(pci2, pni2, fs2, fd2, ones_h, z1)
    cnt3 = jnp.stack([cn, cd, cf], axis=-1)

    hc_lo, hc_hi = pl.pallas_call(
        _t1a_body,
        grid=(23,),
        in_specs=[
            pl.BlockSpec((2176, 16), lambda i: (i, 0)),
            pl.BlockSpec((2176, 1), lambda i: (i, 0)),
            _full((16, 64)),
            _full((1, 64)),
        ],
        out_specs=[
            pl.BlockSpec((2176, 32), lambda i: (i, 0)),
            pl.BlockSpec((2176, 32), lambda i: (i, 0)),
        ],
        out_shape=[
            jax.ShapeDtypeStruct((CP, 32), F32),
            jax.ShapeDtypeStruct((CP, 32), F32),
        ],
    )(cell_p, cc.reshape(CP, 1), p["cell_lin"]["W"],
      p["cell_lin"]["b"].reshape(1, 64))

    hn_src, hn_dst, hv = pl.pallas_call(
        _t1b_body,
        grid=(8,),
        in_specs=[
            pl.BlockSpec((1264, 8), lambda i: (i, 0)),
            pl.BlockSpec((1264, 3), lambda i: (i, 0)),
            _full((8, 64)),
            _full((1, 64)),
            _full((64, 64)),
            _full((1, 64)),
        ],
        out_specs=[
            pl.BlockSpec((1264, 64), lambda i: (i, 0)),
            pl.BlockSpec((1264, 64), lambda i: (i, 0)),
            pl.BlockSpec((1264, 64), lambda i: (i, 0)),
        ],
        out_shape=[
            jax.ShapeDtypeStruct((NPAD, 64), F32),
            jax.ShapeDtypeStruct((NPAD, 64), F32),
            jax.ShapeDtypeStruct((NPAD, 64), F32),
        ],
    )(net_p, cnt3, p["net_lin"]["W"], p["net_lin"]["b"].reshape(1, 64),
      p["cf_node"]["W"], p["cf_node"]["b"].reshape(1, 64))

    he0, he1, he2, he3, s_pin = pl.pallas_call(
        _t1c_body,
        grid=(400,),
        in_specs=[
            pl.BlockSpec((2048, 8), lambda i: (i, 0)),
            _full((8, 16)),
            _full((1, 16)),
            _full((16, 64)),
            _full((1, 64)),
            _full((64, 64)),
            _full((1, 64)),
            _full((16, 2)),
        ],
        out_specs=[
            pl.BlockSpec((2048, 16), lambda i: (i, 0)),
            pl.BlockSpec((2048, 16), lambda i: (i, 0)),
            pl.BlockSpec((2048, 16), lambda i: (i, 0)),
            pl.BlockSpec((2048, 16), lambda i: (i, 0)),
            pl.BlockSpec((2048, 2), lambda i: (i, 0)),
        ],
        out_shape=[
            jax.ShapeDtypeStruct((NPP, 16), F32),
            jax.ShapeDtypeStruct((NPP, 16), F32),
            jax.ShapeDtypeStruct((NPP, 16), F32),
            jax.ShapeDtypeStruct((NPP, 16), F32),
            jax.ShapeDtypeStruct((NPP, 2), F32),
        ],
    )(pin_p, p["pin_lin"]["W"], p["pin_lin"]["b"].reshape(1, 16),
      p["cf_edge1"]["W"], p["cf_edge1"]["b"].reshape(1, 64),
      p["cf_edge2"]["W"], p["cf_edge2"]["b"].reshape(1, 64), wsp)

    fsd2 = jnp.concatenate([fs2, fd2], axis=0)
    hn2 = jnp.concatenate([hn_src, hn_dst], axis=0)
    out2 = _k1(hn2, fsd2, z64)[0]
    accf, accs = out2[:NPAD], out2[NPAD:]
    acc1_lo, acc1_hi = _k2(hc_lo, hc_hi, pci2, pni2, z2d)
    hv0, hv1, hv2, hv3 = (hv[:, :16], hv[:, 16:32], hv[:, 32:48], hv[:, 48:])
    a40, a41, a42, a43 = _k3(hv0, hv1, hv2, hv3, he0, he1, he2, he3,
                             pci2, pni2, z16)

    u = pl.pallas_call(
        _t2a_body,
        grid=(4,),
        in_specs=[
            pl.BlockSpec((2528, 32), lambda i: (i, 0)),
            pl.BlockSpec((2528, 32), lambda i: (i, 0)),
            pl.BlockSpec((2528, 64), lambda i: (i, 0)),
            pl.BlockSpec((2528, 64), lambda i: (i, 0)),
            pl.BlockSpec((2528, 3), lambda i: (i, 0)),
            _full((64, 64)), _full((1, 64)),
            _full((64, 64)), _full((1, 64)),
            _full((64, 64)), _full((1, 64)),
            _full((64, 8)), _full((1, 8)),
        ],
        out_specs=[pl.BlockSpec((2528, 8), lambda i: (i, 0))],
        out_shape=[jax.ShapeDtypeStruct((NPAD, 8), F32)],
    )(acc1_lo, acc1_hi, accf, accs, cnt3,
      p["gc_pins"]["W"], p["gc_pins"]["b"].reshape(1, 64),
      p["gc_father"]["W"], p["gc_father"]["b"].reshape(1, 64),
      p["gc_son"]["W"], p["gc_son"]["b"].reshape(1, 64),
      w6, b6)[0]

    s_cell = pl.pallas_call(
        _t2b_body,
        grid=(23,),
        in_specs=[
            pl.BlockSpec((2176, 16), lambda i: (i, 0)),
            pl.BlockSpec((2176, 16), lambda i: (i, 0)),
            pl.BlockSpec((2176, 16), lambda i: (i, 0)),
            pl.BlockSpec((2176, 16), lambda i: (i, 0)),
            _full((64, 64)), _full((1, 64)), _full((64, 2)),
        ],
        out_specs=[pl.BlockSpec((2176, 2), lambda i: (i, 0))],
        out_shape=[jax.ShapeDtypeStruct((CP, 2), F32)],
    )(a40, a41, a42, a43, p["cf_out"]["W"], p["cf_out"]["b"].reshape(1, 64),
      wsc)[0]

    nd0, na0, pd0, pa0 = _k4(u.reshape(UF), s_cell.reshape(SF),
                             nn0_2, nn1_2, pni2, pci2)

    net_dis2 = pl.pallas_call(
        _t3b_body,
        in_specs=[_full((320, 128))],
        out_specs=[_full((320, 128))],
        out_shape=[jax.ShapeDtypeStruct((320, 128), F32)],
    )(nd0)[0]

    sd = s_pin[:, 0].reshape(PINR, 128)
    sa = s_pin[:, 1].reshape(PINR, 128)
    dis2, ang2 = pl.pallas_call(
        _t3_body,
        grid=(8,),
        in_specs=[pl.BlockSpec((800, 128), lambda i: (i, 0))] * 4,
        out_specs=[pl.BlockSpec((800, 128), lambda i: (i, 0))] * 2,
        out_shape=[
            jax.ShapeDtypeStruct((PINR, 128), F32),
            jax.ShapeDtypeStruct((PINR, 128), F32),
        ],
    )(pd0, pa0, sd, sa)

    return (net_dis2.reshape(-1)[:N_NN], na0.reshape(-1)[:N_NN],
            dis2.reshape(-1)[:N_PIN], ang2.reshape(-1)[:N_PIN])


# speedup vs baseline: 3.7995x; 1.0182x over previous
import functools

import jax
import jax.numpy as jnp
from jax import lax
from jax.experimental import pallas as pl
from jax.experimental.pallas import tpu as pltpu
from jax.experimental.pallas import tpu_sc as plsc

F32 = jnp.float32
I32 = jnp.int32
LOG2 = 0.6931471805599453

N_CELL, N_NET, N_PIN, N_NN = 50000, 10000, 800000, 40000
CP, NPAD = 50048, 10112
CSL, NSL = 3128, 632
NNP = 40960
NNF = 49152
NPP = 819200
PINR = NPP // 128
UF = NPAD * 8
SF = CP * 2

_mesh = plsc.VectorSubcoreMesh(core_axis_name="c", subcore_axis_name="s")


def _softplus(x):
    return jnp.log(1.0 + jnp.exp(jnp.minimum(x, 30.0)))


def _ssp(x):
    return _softplus(x) - LOG2


def _rs(c):
    return lax.rsqrt(jnp.maximum(c, 1.0))


def _wid_axes():
    return lax.axis_index("c"), lax.axis_index("s")


def _fire_drain(descs):
    for d in descs:
        d.wait()


@functools.partial(
    pl.kernel,
    out_type=[
        jax.ShapeDtypeStruct((CP,), F32),
        jax.ShapeDtypeStruct((NPAD,), F32),
        jax.ShapeDtypeStruct((NPAD,), F32),
        jax.ShapeDtypeStruct((NPAD,), F32),
    ],
    mesh=_mesh,
    compiler_params=pltpu.CompilerParams(use_tc_tiling_on_sc=False),
    scratch_types=[
        pltpu.VMEM_SHARED((CP,), F32),
        pltpu.VMEM_SHARED((NPAD,), F32),
        pltpu.VMEM_SHARED((NPAD,), F32),
        pltpu.VMEM_SHARED((NPAD,), F32),
        pltpu.VMEM((16, 128), I32),
        pltpu.VMEM((16, 128), F32),
        pltpu.VMEM((CSL,), F32),
        pltpu.SemaphoreType.DMA,
    ],
)
def _k0(pci2, pni2, fs2, fd2, ones_h, z1, out_cc, out_cn, out_cf, out_cd,
        hc_s, hn_s, hf_s, hd_s, idx_v, ones_v, zb, sem):
    cid, sid = _wid_axes()
    pltpu.sync_copy(ones_h, ones_v)
    pltpu.sync_copy(z1, zb)
    pltpu.sync_copy(zb, hc_s.at[pl.ds(sid * CSL, CSL)])
    pltpu.sync_copy(zb.at[pl.ds(0, NSL)], hn_s.at[pl.ds(sid * NSL, NSL)])
    pltpu.sync_copy(zb.at[pl.ds(0, NSL)], hf_s.at[pl.ds(sid * NSL, NSL)])
    pltpu.sync_copy(zb.at[pl.ds(0, NSL)], hd_s.at[pl.ds(sid * NSL, NSL)])
    plsc.subcore_barrier()

    def pin_hist(src2d, hist):
        def body(k, _):
            c = sid + 16 * k
            pltpu.sync_copy(src2d.at[pl.ds(c * 16, 16)], idx_v)
            _fire_drain([
                pltpu.async_copy(ones_v.at[j], hist.at[idx_v.at[j]], sem,
                                 add=True)
                for j in range(16)
            ])
            return 0

        lax.fori_loop(0, 25, body, 0)

    @pl.when(cid == 0)
    def _():
        pin_hist(pci2, hc_s)

    @pl.when(cid == 1)
    def _():
        pin_hist(pni2, hn_s)

        def body2(k, _):
            c = sid + 16 * k

            @pl.when(c < 48)
            def _():
                r0 = c * 8
                pltpu.sync_copy(fs2.at[pl.ds(r0, 8)], idx_v.at[pl.ds(0, 8)])
                _fire_drain([
                    pltpu.async_copy(ones_v.at[j], hf_s.at[idx_v.at[j]], sem,
                                     add=True)
                    for j in range(8)
                ])
                pltpu.sync_copy(fd2.at[pl.ds(r0, 8)], idx_v.at[pl.ds(0, 8)])
                _fire_drain([
                    pltpu.async_copy(ones_v.at[j], hd_s.at[idx_v.at[j]], sem,
                                     add=True)
                    for j in range(8)
                ])
            return 0

        lax.fori_loop(0, 3, body2, 0)

    plsc.subcore_barrier()

    @pl.when(cid == 0)
    def _():
        pltpu.sync_copy(hc_s.at[pl.ds(sid * CSL, CSL)], zb)
        pltpu.sync_copy(zb, out_cc.at[pl.ds(sid * CSL, CSL)])

    @pl.when(cid == 1)
    def _():
        s0 = sid * NSL
        pltpu.sync_copy(hn_s.at[pl.ds(s0, NSL)], zb.at[pl.ds(0, NSL)])
        pltpu.sync_copy(zb.at[pl.ds(0, NSL)], out_cn.at[pl.ds(s0, NSL)])
        pltpu.sync_copy(hf_s.at[pl.ds(s0, NSL)], zb.at[pl.ds(0, NSL)])
        pltpu.sync_copy(zb.at[pl.ds(0, NSL)], out_cf.at[pl.ds(s0, NSL)])
        pltpu.sync_copy(hd_s.at[pl.ds(s0, NSL)], zb.at[pl.ds(0, NSL)])
        pltpu.sync_copy(zb.at[pl.ds(0, NSL)], out_cd.at[pl.ds(s0, NSL)])


@functools.partial(
    pl.kernel,
    out_type=[jax.ShapeDtypeStruct((2 * NPAD, 64), F32)],
    mesh=_mesh,
    compiler_params=pltpu.CompilerParams(use_tc_tiling_on_sc=False),
    scratch_types=[
        pltpu.VMEM_SHARED((NPAD, 64), F32),
        pltpu.VMEM((8, 128), I32),
        pltpu.VMEM((8, 128), I32),
        pltpu.VMEM((1024, 64), F32),
        pltpu.SemaphoreType.DMA,
        pltpu.SemaphoreType.DMA,
    ],
)
def _k1(hn2, fsd2, z64, out2,
        acc_s, idxg, idxs, rows, semg, sems):
    cid, sid = _wid_axes()
    s0 = sid * NSL

    stg = rows.at[pl.ds(0, NSL), :]
    pltpu.sync_copy(z64, stg)
    pltpu.sync_copy(stg, acc_s.at[pl.ds(s0, NSL)])
    plsc.subcore_barrier()

    def body(k, _):
        c = sid + 16 * k
        r0 = c * 8
        pltpu.sync_copy(fsd2.at[pl.ds(cid * 384 + r0, 8)], idxg)
        pltpu.sync_copy(fsd2.at[pl.ds((1 - cid) * 384 + r0, 8)], idxs)

        off = cid * NPAD

        @plsc.parallel_loop(0, 64, 1, unroll=8)
        def _(i):
            r = i // 8
            cc = (i % 8) * 16
            idxg[r, pl.ds(cc, 16)] = idxg[r, pl.ds(cc, 16)] + off

        _fire_drain([
            pltpu.async_copy(hn2.at[idxg.at[j]],
                             rows.at[pl.ds(j * 128, 128)], semg)
            for j in range(8)
        ])
        _fire_drain([
            pltpu.async_copy(rows.at[pl.ds(j * 128, 128)],
                             acc_s.at[idxs.at[j]], sems, add=True)
            for j in range(8)
        ])
        return 0

    lax.fori_loop(0, 3, body, 0)
    plsc.subcore_barrier()

    pltpu.sync_copy(acc_s.at[pl.ds(s0, NSL)], stg)
    pltpu.sync_copy(stg, out2.at[pl.ds(cid * NPAD + s0, NSL)])


@functools.partial(
    pl.kernel,
    out_type=[jax.ShapeDtypeStruct((2 * NPAD, 32), F32)],
    mesh=_mesh,
    compiler_params=pltpu.CompilerParams(use_tc_tiling_on_sc=False),
    scratch_types=[
        pltpu.VMEM_SHARED((NPAD, 32), F32),
        pltpu.VMEM((8, 128), I32),
        pltpu.VMEM((8, 128), I32),
        pltpu.VMEM((8, 128), I32),
        pltpu.VMEM((8, 128), I32),
        pltpu.VMEM((1024, 32), F32),
        pltpu.VMEM((1024, 32), F32),
        pltpu.SemaphoreType.DMA,
        pltpu.SemaphoreType.DMA,
        pltpu.SemaphoreType.DMA,
    ],
)
def _k2(hc2, pci2, pni2, z2d, out2,
        acc_s, idxg0, idxg1, idxs0, idxs1, rows0, rows1,
        semg, sems0, sems1):
    cid, sid = _wid_axes()
    s0 = sid * NSL
    goff = cid * CP

    pltpu.sync_copy(z2d.at[pl.ds(0, 1024)], rows0)
    pltpu.sync_copy(z2d.at[pl.ds(0, 1024)], rows1)
    pltpu.sync_copy(rows0.at[pl.ds(0, NSL), :], acc_s.at[pl.ds(s0, NSL)])
    plsc.subcore_barrier()

    def drain_scat(sems):
        pltpu.make_async_copy(hc2.at[pl.ds(0, 1024)], rows0, sems).wait()

    pltpu.sync_copy(pni2.at[pl.ds(0, 8)], idxs0)
    pltpu.sync_copy(pni2.at[pl.ds(0, 8)], idxs1)
    for j in range(8):
        pltpu.async_copy(rows0.at[pl.ds(j * 128, 128)],
                         acc_s.at[idxs0.at[j]], sems0, add=True)
    for j in range(8):
        pltpu.async_copy(rows1.at[pl.ds(j * 128, 128)],
                         acc_s.at[idxs1.at[j]], sems1, add=True)

    def do_chunk(k, idxg, idxs, rows, sems):
        c = sid + 16 * k
        r0 = c * 8
        pltpu.sync_copy(pci2.at[pl.ds(r0, 8)], idxg)
        pltpu.sync_copy(pni2.at[pl.ds(r0, 8)], idxs)

        @plsc.parallel_loop(0, 64, 1, unroll=8)
        def _(i):
            r = i // 8
            cc = (i % 8) * 16
            idxg[r, pl.ds(cc, 16)] = idxg[r, pl.ds(cc, 16)] + goff

        drain_scat(sems)
        for j in range(8):
            pltpu.async_copy(hc2.at[idxg.at[j]],
                             rows.at[pl.ds(j * 128, 128)], semg)
        pltpu.make_async_copy(hc2.at[pl.ds(0, 1024)], rows0, semg).wait()
        for j in range(8):
            pltpu.async_copy(rows.at[pl.ds(j * 128, 128)],
                             acc_s.at[idxs.at[j]], sems, add=True)

    def body(k, _):
        do_chunk(2 * k, idxg0, idxs0, rows0, sems0)
        do_chunk(2 * k + 1, idxg1, idxs1, rows1, sems1)
        return 0

    lax.fori_loop(0, 25, body, 0)
    drain_scat(sems0)
    drain_scat(sems1)
    plsc.subcore_barrier()

    stgo = rows0.at[pl.ds(0, NSL), :]
    pltpu.sync_copy(acc_s.at[pl.ds(s0, NSL)], stgo)
    pltpu.sync_copy(stgo, out2.at[pl.ds(cid * NPAD + s0, NSL)])


@functools.partial(
    pl.kernel,
    out_type=[jax.ShapeDtypeStruct((CP, 16), F32) for _ in range(4)],
    mesh=_mesh,
    compiler_params=pltpu.CompilerParams(use_tc_tiling_on_sc=False),
    scratch_types=[
        pltpu.VMEM_SHARED((CP, 16), F32),
        pltpu.VMEM_SHARED((NPAD, 16), F32),
        pltpu.VMEM((8, 128), I32),
        pltpu.VMEM((8, 128), I32),
        pltpu.VMEM((8, 128), I32),
        pltpu.VMEM((1024, 16), F32),
        pltpu.VMEM((1024, 16), F32),
        pltpu.SemaphoreType.DMA,
        pltpu.SemaphoreType.DMA,
        pltpu.SemaphoreType.DMA,
    ],
)
def _k3(hv0, hv1, hv2, hv3, he0, he1, he2, he3, pci2, pni2, z16,
        out0, out1, out2, out3,
        acc_s, tab_s, idxg, idxs0, idxs1, rows, heb, semg, sems, semh):
    cid, sid = _wid_axes()
    c0 = sid * CSL
    s0 = sid * NSL

    def one_pass(hvq, heq, outq):
        stg = rows.at[pl.ds(0, NSL), :]
        pltpu.sync_copy(hvq.at[pl.ds(s0, NSL)], stg)
        pltpu.sync_copy(stg, tab_s.at[pl.ds(s0, NSL)])
        for off, n in ((0, 1024), (1024, 1024), (2048, 1024), (3072, 56)):
            stgz = rows.at[pl.ds(0, n), :]
            pltpu.sync_copy(z16.at[pl.ds(off, n)], stgz)
            pltpu.sync_copy(stgz, acc_s.at[pl.ds(c0 + off, n)])
        plsc.subcore_barrier()

        def drain_prev():
            pltpu.make_async_copy(heq.at[pl.ds(0, 1024), :], heb,
                                  sems).wait()

        pltpu.sync_copy(pci2.at[pl.ds(0, 8)], idxs1)
        for j in range(8):
            pltpu.async_copy(rows.at[pl.ds(j * 128, 128), :],
                             acc_s.at[idxs1.at[j]], sems, add=True)

        def do_chunk(k, idxs):
            c = sid + 16 * k
            r0 = c * 8
            pltpu.sync_copy(pni2.at[pl.ds(r0, 8)], idxg)
            pltpu.sync_copy(pci2.at[pl.ds(r0, 8)], idxs)
            hed = pltpu.async_copy(heq.at[pl.ds(c * 1024, 1024), :], heb,
                                   semh)
            drain_prev()
            _fire_drain([
                pltpu.async_copy(tab_s.at[idxg.at[j]],
                                 rows.at[pl.ds(j * 128, 128)], semg)
                for j in range(8)
            ])
            hed.wait()

            @plsc.parallel_loop(0, 1024, 1, unroll=8)
            def _(q):
                rows[q, pl.ds(0, 16)] = (heb[q, pl.ds(0, 16)] *
                                         rows[q, pl.ds(0, 16)])

            for j in range(8):
                pltpu.async_copy(rows.at[pl.ds(j * 128, 128), :],
                                 acc_s.at[idxs.at[j]], sems, add=True)

        def body(j, _):
            do_chunk(2 * j, idxs0)
            do_chunk(2 * j + 1, idxs1)
            return 0

        lax.fori_loop(0, 25, body, 0)
        drain_prev()
        plsc.subcore_barrier()

        for off, n in ((0, 1024), (1024, 1024), (2048, 1024), (3072, 56)):
            stgo = rows.at[pl.ds(0, n), :]
            pltpu.sync_copy(acc_s.at[pl.ds(c0 + off, n)], stgo)
            pltpu.sync_copy(stgo, outq.at[pl.ds(c0 + off, n)])
        plsc.subcore_barrier()

    @pl.when(cid == 0)
    def _():
        one_pass(hv0, he0, out0)
        one_pass(hv1, he1, out1)

    @pl.when(cid == 1)
    def _():
        one_pass(hv2, he2, out2)
        one_pass(hv3, he3, out3)


@functools.partial(
    pl.kernel,
    out_type=[
        jax.ShapeDtypeStruct((320, 128), F32),
        jax.ShapeDtypeStruct((320, 128), F32),
        jax.ShapeDtypeStruct((PINR, 128), F32),
        jax.ShapeDtypeStruct((PINR, 128), F32),
    ],
    mesh=_mesh,
    compiler_params=pltpu.CompilerParams(use_tc_tiling_on_sc=False),
    scratch_types=[
        pltpu.VMEM_SHARED((UF,), F32),
        pltpu.VMEM_SHARED((SF,), F32),
        pltpu.VMEM((16, 128), I32),
        pltpu.VMEM((16, 128), I32),
        pltpu.VMEM((16, 128), I32),
        pltpu.VMEM((16, 128), I32),
        pltpu.VMEM((16, 128), I32),
        pltpu.VMEM((16, 128), I32),
        pltpu.VMEM((16, 128), F32),
        pltpu.VMEM((16, 128), F32),
        pltpu.VMEM((16, 128), F32),
        pltpu.VMEM((16, 128), F32),
        pltpu.VMEM((SF // 16,), F32),
        pltpu.SemaphoreType.DMA,
    ],
)
def _k4(uf, scf, nn0_2, nn1_2, pni2, pci2, nd0, na0, pd0, pa0,
        u_s, sc_s, ia, ib, f0, f1, f2, f3, g0, g1, g2, g3, stb, sem):
    cid, sid = _wid_axes()
    pltpu.sync_copy(uf.at[pl.ds(sid * (UF // 16), UF // 16)],
                    stb.at[pl.ds(0, UF // 16)])
    pltpu.sync_copy(stb.at[pl.ds(0, UF // 16)],
                    u_s.at[pl.ds(sid * (UF // 16), UF // 16)])
    pltpu.sync_copy(scf.at[pl.ds(sid * (SF // 16), SF // 16)], stb)
    pltpu.sync_copy(stb, sc_s.at[pl.ds(sid * (SF // 16), SF // 16)])
    plsc.subcore_barrier()

    @pl.when(cid == 0)
    def _():
        def nbody(k, _):
            c = sid + 16 * k

            @pl.when(c < 40)
            def _():
                r0 = c * 8
                pltpu.sync_copy(nn0_2.at[pl.ds(r0, 8)], ia.at[pl.ds(0, 8)])
                pltpu.sync_copy(nn1_2.at[pl.ds(r0, 8)], ib.at[pl.ds(0, 8)])

                @plsc.parallel_loop(0, 64, 1, unroll=8)
                def _(i):
                    r = i // 8
                    cc = (i % 8) * 16
                    va = ia[r, pl.ds(cc, 16)] * 8
                    vb = ib[r, pl.ds(cc, 16)] * 8
                    f0[r, pl.ds(cc, 16)] = va
                    f1[r, pl.ds(cc, 16)] = vb + 1
                    f2[r, pl.ds(cc, 16)] = va + 2
                    f3[r, pl.ds(cc, 16)] = vb + 3

                _fire_drain(
                    [pltpu.async_copy(u_s.at[f0.at[j]], g0.at[j], sem)
                     for j in range(8)] +
                    [pltpu.async_copy(u_s.at[f1.at[j]], g1.at[j], sem)
                     for j in range(8)] +
                    [pltpu.async_copy(u_s.at[f2.at[j]], g2.at[j], sem)
                     for j in range(8)] +
                    [pltpu.async_copy(u_s.at[f3.at[j]], g3.at[j], sem)
                     for j in range(8)])

                @plsc.parallel_loop(0, 64, 1, unroll=8)
                def _(i):
                    r = i // 8
                    cc = (i % 8) * 16
                    g0[r, pl.ds(cc, 16)] = (g0[r, pl.ds(cc, 16)] +
                                            g1[r, pl.ds(cc, 16)])
                    g2[r, pl.ds(cc, 16)] = (g2[r, pl.ds(cc, 16)] +
                                            g3[r, pl.ds(cc, 16)])

                pltpu.sync_copy(g0.at[pl.ds(0, 8)], nd0.at[pl.ds(r0, 8)])
                pltpu.sync_copy(g2.at[pl.ds(0, 8)], na0.at[pl.ds(r0, 8)])
            return 0

        lax.fori_loop(0, 3, nbody, 0)

    def pbody(k, _):
        ci = sid + 16 * k

        @pl.when(ci < 200)
        def _():
            c = 2 * ci + cid
            r0 = c * 16
            pltpu.sync_copy(pni2.at[pl.ds(r0, 16)], ia)
            pltpu.sync_copy(pci2.at[pl.ds(r0, 16)], ib)

            @plsc.parallel_loop(0, 128, 1, unroll=8)
            def _(i):
                r = i // 8
                cc = (i % 8) * 16
                va = ia[r, pl.ds(cc, 16)] * 8
                vb = ib[r, pl.ds(cc, 16)] * 2
                f0[r, pl.ds(cc, 16)] = va + 4
                f1[r, pl.ds(cc, 16)] = va + 5
                f2[r, pl.ds(cc, 16)] = vb
                f3[r, pl.ds(cc, 16)] = vb + 1

            _fire_drain(
                [pltpu.async_copy(u_s.at[f0.at[j]], g0.at[j], sem)
                 for j in range(16)] +
                [pltpu.async_copy(u_s.at[f1.at[j]], g1.at[j], sem)
                 for j in range(16)] +
                [pltpu.async_copy(sc_s.at[f2.at[j]], g2.at[j], sem)
                 for j in range(16)] +
                [pltpu.async_copy(sc_s.at[f3.at[j]], g3.at[j], sem)
                 for j in range(16)])

            @plsc.parallel_loop(0, 128, 1, unroll=8)
            def _(i):
                r = i // 8
                cc = (i % 8) * 16
                g0[r, pl.ds(cc, 16)] = (g0[r, pl.ds(cc, 16)] +
                                        g2[r, pl.ds(cc, 16)])
                g1[r, pl.ds(cc, 16)] = (g1[r, pl.ds(cc, 16)] +
                                        g3[r, pl.ds(cc, 16)])

            pltpu.sync_copy(g0, pd0.at[pl.ds(r0, 16)])
            pltpu.sync_copy(g1, pa0.at[pl.ds(r0, 16)])
        return 0

    lax.fori_loop(0, 13, pbody, 0)


def _t1a_body(x_ref, cnt_ref, w_ref, b_ref, lo_ref, hi_ref):
    h = jnp.tanh(jnp.dot(x_ref[...], w_ref[...],
                         preferred_element_type=F32) + b_ref[...])
    h = h * _rs(cnt_ref[...])
    lo_ref[...] = h[:, :32]
    hi_ref[...] = h[:, 32:]


def _t1b_body(x_ref, cnt3_ref, w_ref, b_ref, cw_ref, cb_ref,
              hs_ref, hd_ref, lo_ref):
    hn = jnp.tanh(jnp.dot(x_ref[...], w_ref[...],
                          preferred_element_type=F32) + b_ref[...])
    cnt3 = cnt3_ref[...]
    hs_ref[...] = hn * _rs(cnt3[:, 2:3])
    hd_ref[...] = hn * _rs(cnt3[:, 1:2])
    hv = jnp.dot(hn, cw_ref[...], preferred_element_type=F32) + cb_ref[...]
    lo_ref[...] = hv


def _t1c_body(x_ref, w1_ref, b1_ref, w2_ref, b2_ref, w3_ref, b3_ref, wsp_ref,
              q0_ref, q1_ref, q2_ref, q3_ref, sp_ref):
    hp = jnp.tanh(jnp.dot(x_ref[...], w1_ref[...],
                          preferred_element_type=F32) + b1_ref[...])
    t = _ssp(jnp.dot(hp, w2_ref[...], preferred_element_type=F32) + b2_ref[...])
    he = _ssp(jnp.dot(t, w3_ref[...], preferred_element_type=F32) + b3_ref[...])
    q0_ref[...] = he[:, :16]
    q1_ref[...] = he[:, 16:32]
    q2_ref[...] = he[:, 32:48]
    q3_ref[...] = he[:, 48:]
    sp_ref[...] = jnp.dot(hp, wsp_ref[...], preferred_element_type=F32)


def _t2a_body(a1l_ref, a1h_ref, af_ref, as_ref, cnt3_ref,
              wp_ref, bp_ref, wf_ref, bf_ref, ws_ref, bs_ref, w6_ref, b6_ref,
              u_ref):
    cnt3 = cnt3_ref[...]
    a1 = jnp.concatenate([a1l_ref[...], a1h_ref[...]], axis=-1)
    op = jnp.dot(a1 * _rs(cnt3[:, 0:1]), wp_ref[...],
                 preferred_element_type=F32) + bp_ref[...]
    of = jnp.dot(af_ref[...] * _rs(cnt3[:, 1:2]), wf_ref[...],
                 preferred_element_type=F32) + bf_ref[...]
    os_ = jnp.dot(as_ref[...] * _rs(cnt3[:, 2:3]), ws_ref[...],
                  preferred_element_type=F32) + bs_ref[...]
    h = jnp.maximum(jnp.maximum(op, of), os_)
    u_ref[...] = (jnp.dot(h, w6_ref[...], preferred_element_type=F32) +
                  b6_ref[...])


def _t2b_body(a40_ref, a41_ref, a42_ref, a43_ref, wo_ref, bo_ref, wsc_ref,
              sc_ref):
    a4 = jnp.concatenate([a40_ref[...], a41_ref[...], a42_ref[...],
                          a43_ref[...]], axis=-1)
    h = _ssp(jnp.dot(a4, wo_ref[...], preferred_element_type=F32) + bo_ref[...])
    sc_ref[...] = jnp.dot(h, wsc_ref[...], preferred_element_type=F32)


def _t3_body(pd_ref, pa_ref, sd_ref, sa_ref, dis_ref, ang_ref):
    dis_ref[...] = _softplus(pd_ref[...] + sd_ref[...])
    ang_ref[...] = pa_ref[...] + sa_ref[...]


def _t3b_body(nd_ref, dis_ref):
    dis_ref[...] = _softplus(nd_ref[...])


def _full(shape):
    nd = len(shape)
    return pl.BlockSpec(shape, lambda *_: (0,) * nd)


def kernel(cell_feat, net_feat, pin_feat, pin_cell_idx, pin_net_idx,
           father_src, father_dst, net_net_pair, params):
    p = params
    pci = pin_cell_idx.astype(I32)
    pni = pin_net_idx.astype(I32)
    fs = father_src.astype(I32)
    fd = father_dst.astype(I32)
    nn0 = net_net_pair[:, 0].astype(I32)
    nn1 = net_net_pair[:, 1].astype(I32)

    dead_n = 10048 + (jnp.arange(NNP - N_NN, dtype=I32) % 64)
    dead_f = 10048 + (jnp.arange(NNF - N_NN, dtype=I32) % 64)
    dead_np = 10048 + (jnp.arange(NPP - N_PIN, dtype=I32) % 64)
    dead_cp = 50000 + (jnp.arange(NPP - N_PIN, dtype=I32) % 48)
    pci2 = jnp.concatenate([pci, dead_cp]).reshape(PINR, 128)
    pni2 = jnp.concatenate([pni, dead_np]).reshape(PINR, 128)
    fs2 = jnp.concatenate([fs, dead_f]).reshape(384, 128)
    fd2 = jnp.concatenate([fd, dead_f]).reshape(384, 128)
    nn0_2 = jnp.concatenate([nn0, dead_n]).reshape(320, 128)
    nn1_2 = jnp.concatenate([nn1, dead_n]).reshape(320, 128)

    ones_h = jnp.ones((16, 128), F32)
    z1 = jnp.zeros((CSL,), F32)
    z2d = jnp.zeros((CSL, 32), F32)
    z16 = jnp.zeros((CSL, 16), F32)
    z64 = jnp.zeros((NSL, 64), F32)

    cell_p = jnp.pad(cell_feat, ((0, CP - N_CELL), (0, 0)))
    net_p = jnp.pad(net_feat, ((0, NPAD - N_NET), (0, 0)))
    pin_p = jnp.pad(pin_feat, ((0, NPP - N_PIN), (0, 0)))

    wd, wa = p["net_dis"]["W"][:, 0], p["net_angle"]["W"][:, 0]
    wpd, wpa = p["pin_dis"]["W"][:, 0], p["pin_angle"]["W"][:, 0]
    zc = jnp.zeros((64,), F32)
    w6 = jnp.stack([wd[:64], wd[64:], wa[:64], wa[64:], wpd[:64], wpa[:64],
                    zc, zc], axis=-1)
    e = jnp.eye(8, dtype=F32)
    b6 = (e[0] * p["net_dis"]["b"][0] + e[2] * p["net_angle"]["b"][0] +
          e[4] * p["pin_dis"]["b"][0] + e[5] * p["pin_angle"]["b"][0])
    b6 = b6.reshape(1, 8)
    wsp = jnp.stack([wpd[64:80], wpa[64:80]], axis=-1)
    wsc = jnp.stack([wpd[80:], wpa[80:]], axis=-1)

    cc, cn, cf, cd = _k0(pci2, pni2, fs2, fd2, ones_h, z1)
    cnt3 = jnp.stack([cn, cd, cf], axis=-1)

    hc_lo, hc_hi = pl.pallas_call(
        _t1a_body,
        grid=(23,),
        in_specs=[
            pl.BlockSpec((2176, 16), lambda i: (i, 0)),
            pl.BlockSpec((2176, 1), lambda i: (i, 0)),
            _full((16, 64)),
            _full((1, 64)),
        ],
        out_specs=[
            pl.BlockSpec((2176, 32), lambda i: (i, 0)),
            pl.BlockSpec((2176, 32), lambda i: (i, 0)),
        ],
        out_shape=[
            jax.ShapeDtypeStruct((CP, 32), F32),
            jax.ShapeDtypeStruct((CP, 32), F32),
        ],
    )(cell_p, cc.reshape(CP, 1), p["cell_lin"]["W"],
      p["cell_lin"]["b"].reshape(1, 64))

    hn_src, hn_dst, hv = pl.pallas_call(
        _t1b_body,
        grid=(8,),
        in_specs=[
            pl.BlockSpec((1264, 8), lambda i: (i, 0)),
            pl.BlockSpec((1264, 3), lambda i: (i, 0)),
            _full((8, 64)),
            _full((1, 64)),
            _full((64, 64)),
            _full((1, 64)),
        ],
        out_specs=[
            pl.BlockSpec((1264, 64), lambda i: (i, 0)),
            pl.BlockSpec((1264, 64), lambda i: (i, 0)),
            pl.BlockSpec((1264, 64), lambda i: (i, 0)),
        ],
        out_shape=[
            jax.ShapeDtypeStruct((NPAD, 64), F32),
            jax.ShapeDtypeStruct((NPAD, 64), F32),
            jax.ShapeDtypeStruct((NPAD, 64), F32),
        ],
    )(net_p, cnt3, p["net_lin"]["W"], p["net_lin"]["b"].reshape(1, 64),
      p["cf_node"]["W"], p["cf_node"]["b"].reshape(1, 64))

    he0, he1, he2, he3, s_pin = pl.pallas_call(
        _t1c_body,
        grid=(200,),
        in_specs=[
            pl.BlockSpec((4096, 8), lambda i: (i, 0)),
            _full((8, 16)),
            _full((1, 16)),
            _full((16, 64)),
            _full((1, 64)),
            _full((64, 64)),
            _full((1, 64)),
            _full((16, 2)),
        ],
        out_specs=[
            pl.BlockSpec((4096, 16), lambda i: (i, 0)),
            pl.BlockSpec((4096, 16), lambda i: (i, 0)),
            pl.BlockSpec((4096, 16), lambda i: (i, 0)),
            pl.BlockSpec((4096, 16), lambda i: (i, 0)),
            pl.BlockSpec((4096, 2), lambda i: (i, 0)),
        ],
        out_shape=[
            jax.ShapeDtypeStruct((NPP, 16), F32),
            jax.ShapeDtypeStruct((NPP, 16), F32),
            jax.ShapeDtypeStruct((NPP, 16), F32),
            jax.ShapeDtypeStruct((NPP, 16), F32),
            jax.ShapeDtypeStruct((NPP, 2), F32),
        ],
    )(pin_p, p["pin_lin"]["W"], p["pin_lin"]["b"].reshape(1, 16),
      p["cf_edge1"]["W"], p["cf_edge1"]["b"].reshape(1, 64),
      p["cf_edge2"]["W"], p["cf_edge2"]["b"].reshape(1, 64), wsp)

    fsd2 = jnp.concatenate([fs2, fd2], axis=0)
    hn2 = jnp.concatenate([hn_src, hn_dst], axis=0)
    out2 = _k1(hn2, fsd2, z64)[0]
    accf, accs = out2[:NPAD], out2[NPAD:]
    hc2 = jnp.concatenate([hc_lo, hc_hi], axis=0)
    outk2 = _k2(hc2, pci2, pni2, z2d)[0]
    acc1_lo, acc1_hi = outk2[:NPAD], outk2[NPAD:]
    hv0, hv1, hv2, hv3 = (hv[:, :16], hv[:, 16:32], hv[:, 32:48], hv[:, 48:])
    a40, a41, a42, a43 = _k3(hv0, hv1, hv2, hv3, he0, he1, he2, he3,
                             pci2, pni2, z16)

    u = pl.pallas_call(
        _t2a_body,
        grid=(4,),
        in_specs=[
            pl.BlockSpec((2528, 32), lambda i: (i, 0)),
            pl.BlockSpec((2528, 32), lambda i: (i, 0)),
            pl.BlockSpec((2528, 64), lambda i: (i, 0)),
            pl.BlockSpec((2528, 64), lambda i: (i, 0)),
            pl.BlockSpec((2528, 3), lambda i: (i, 0)),
            _full((64, 64)), _full((1, 64)),
            _full((64, 64)), _full((1, 64)),
            _full((64, 64)), _full((1, 64)),
            _full((64, 8)), _full((1, 8)),
        ],
        out_specs=[pl.BlockSpec((2528, 8), lambda i: (i, 0))],
        out_shape=[jax.ShapeDtypeStruct((NPAD, 8), F32)],
    )(acc1_lo, acc1_hi, accf, accs, cnt3,
      p["gc_pins"]["W"], p["gc_pins"]["b"].reshape(1, 64),
      p["gc_father"]["W"], p["gc_father"]["b"].reshape(1, 64),
      p["gc_son"]["W"], p["gc_son"]["b"].reshape(1, 64),
      w6, b6)[0]

    s_cell = pl.pallas_call(
        _t2b_body,
        grid=(23,),
        in_specs=[
            pl.BlockSpec((2176, 16), lambda i: (i, 0)),
            pl.BlockSpec((2176, 16), lambda i: (i, 0)),
            pl.BlockSpec((2176, 16), lambda i: (i, 0)),
            pl.BlockSpec((2176, 16), lambda i: (i, 0)),
            _full((64, 64)), _full((1, 64)), _full((64, 2)),
        ],
        out_specs=[pl.BlockSpec((2176, 2), lambda i: (i, 0))],
        out_shape=[jax.ShapeDtypeStruct((CP, 2), F32)],
    )(a40, a41, a42, a43, p["cf_out"]["W"], p["cf_out"]["b"].reshape(1, 64),
      wsc)[0]

    nd0, na0, pd0, pa0 = _k4(u.reshape(UF), s_cell.reshape(SF),
                             nn0_2, nn1_2, pni2, pci2)

    net_dis2 = pl.pallas_call(
        _t3b_body,
        in_specs=[_full((320, 128))],
        out_specs=[_full((320, 128))],
        out_shape=[jax.ShapeDtypeStruct((320, 128), F32)],
    )(nd0)[0]

    sd = s_pin[:, 0].reshape(PINR, 128)
    sa = s_pin[:, 1].reshape(PINR, 128)
    dis2, ang2 = pl.pallas_call(
        _t3_body,
        grid=(8,),
        in_specs=[pl.BlockSpec((800, 128), lambda i: (i, 0))] * 4,
        out_specs=[pl.BlockSpec((800, 128), lambda i: (i, 0))] * 2,
        out_shape=[
            jax.ShapeDtypeStruct((PINR, 128), F32),
            jax.ShapeDtypeStruct((PINR, 128), F32),
        ],
    )(pd0, pa0, sd, sa)

    return (net_dis2.reshape(-1)[:N_NN], na0.reshape(-1)[:N_NN],
            dis2.reshape(-1)[:N_PIN], ang2.reshape(-1)[:N_PIN])


# speedup vs baseline: 3.8123x; 1.0034x over previous
import functools

import jax
import jax.numpy as jnp
from jax import lax
from jax.experimental import pallas as pl
from jax.experimental.pallas import tpu as pltpu
from jax.experimental.pallas import tpu_sc as plsc

F32 = jnp.float32
I32 = jnp.int32
LOG2 = 0.6931471805599453

N_CELL, N_NET, N_PIN, N_NN = 50000, 10000, 800000, 40000
CP, NPAD = 50048, 10112
CSL, NSL = 3128, 632
NNP = 40960
NNF = 49152
NPP = 819200
PINR = NPP // 128
UF = NPAD * 8
SF = CP * 2

_mesh = plsc.VectorSubcoreMesh(core_axis_name="c", subcore_axis_name="s")


def _softplus(x):
    return jnp.log(1.0 + jnp.exp(jnp.minimum(x, 30.0)))


def _ssp(x):
    return _softplus(x) - LOG2


def _rs(c):
    return lax.rsqrt(jnp.maximum(c, 1.0))


def _wid_axes():
    return lax.axis_index("c"), lax.axis_index("s")


def _fire_drain(descs):
    for d in descs:
        d.wait()


@functools.partial(
    pl.kernel,
    out_type=[
        jax.ShapeDtypeStruct((CP,), F32),
        jax.ShapeDtypeStruct((NPAD,), F32),
        jax.ShapeDtypeStruct((NPAD,), F32),
        jax.ShapeDtypeStruct((NPAD,), F32),
    ],
    mesh=_mesh,
    compiler_params=pltpu.CompilerParams(use_tc_tiling_on_sc=False),
    scratch_types=[
        pltpu.VMEM_SHARED((CP,), F32),
        pltpu.VMEM_SHARED((NPAD,), F32),
        pltpu.VMEM_SHARED((NPAD,), F32),
        pltpu.VMEM_SHARED((NPAD,), F32),
        pltpu.VMEM((16, 128), I32),
        pltpu.VMEM((16, 128), F32),
        pltpu.VMEM((CSL,), F32),
        pltpu.SemaphoreType.DMA,
    ],
)
def _k0(pci2, pni2, fs2, fd2, ones_h, z1, out_cc, out_cn, out_cf, out_cd,
        hc_s, hn_s, hf_s, hd_s, idx_v, ones_v, zb, sem):
    cid, sid = _wid_axes()
    pltpu.sync_copy(ones_h, ones_v)
    pltpu.sync_copy(z1, zb)
    pltpu.sync_copy(zb, hc_s.at[pl.ds(sid * CSL, CSL)])
    pltpu.sync_copy(zb.at[pl.ds(0, NSL)], hn_s.at[pl.ds(sid * NSL, NSL)])
    pltpu.sync_copy(zb.at[pl.ds(0, NSL)], hf_s.at[pl.ds(sid * NSL, NSL)])
    pltpu.sync_copy(zb.at[pl.ds(0, NSL)], hd_s.at[pl.ds(sid * NSL, NSL)])
    plsc.subcore_barrier()

    def pin_hist(src2d, hist):
        def body(k, _):
            c = sid + 16 * k
            pltpu.sync_copy(src2d.at[pl.ds(c * 16, 16)], idx_v)
            _fire_drain([
                pltpu.async_copy(ones_v.at[j], hist.at[idx_v.at[j]], sem,
                                 add=True)
                for j in range(16)
            ])
            return 0

        lax.fori_loop(0, 25, body, 0)

    @pl.when(cid == 0)
    def _():
        pin_hist(pci2, hc_s)

    @pl.when(cid == 1)
    def _():
        pin_hist(pni2, hn_s)

        def body2(k, _):
            c = sid + 16 * k

            @pl.when(c < 48)
            def _():
                r0 = c * 8
                pltpu.sync_copy(fs2.at[pl.ds(r0, 8)], idx_v.at[pl.ds(0, 8)])
                _fire_drain([
                    pltpu.async_copy(ones_v.at[j], hf_s.at[idx_v.at[j]], sem,
                                     add=True)
                    for j in range(8)
                ])
                pltpu.sync_copy(fd2.at[pl.ds(r0, 8)], idx_v.at[pl.ds(0, 8)])
                _fire_drain([
                    pltpu.async_copy(ones_v.at[j], hd_s.at[idx_v.at[j]], sem,
                                     add=True)
                    for j in range(8)
                ])
            return 0

        lax.fori_loop(0, 3, body2, 0)

    plsc.subcore_barrier()

    @pl.when(cid == 0)
    def _():
        pltpu.sync_copy(hc_s.at[pl.ds(sid * CSL, CSL)], zb)
        pltpu.sync_copy(zb, out_cc.at[pl.ds(sid * CSL, CSL)])

    @pl.when(cid == 1)
    def _():
        s0 = sid * NSL
        pltpu.sync_copy(hn_s.at[pl.ds(s0, NSL)], zb.at[pl.ds(0, NSL)])
        pltpu.sync_copy(zb.at[pl.ds(0, NSL)], out_cn.at[pl.ds(s0, NSL)])
        pltpu.sync_copy(hf_s.at[pl.ds(s0, NSL)], zb.at[pl.ds(0, NSL)])
        pltpu.sync_copy(zb.at[pl.ds(0, NSL)], out_cf.at[pl.ds(s0, NSL)])
        pltpu.sync_copy(hd_s.at[pl.ds(s0, NSL)], zb.at[pl.ds(0, NSL)])
        pltpu.sync_copy(zb.at[pl.ds(0, NSL)], out_cd.at[pl.ds(s0, NSL)])


@functools.partial(
    pl.kernel,
    out_type=[jax.ShapeDtypeStruct((2 * NPAD, 64), F32)],
    mesh=_mesh,
    compiler_params=pltpu.CompilerParams(use_tc_tiling_on_sc=False),
    scratch_types=[
        pltpu.VMEM_SHARED((NPAD, 64), F32),
        pltpu.VMEM((8, 128), I32),
        pltpu.VMEM((8, 128), I32),
        pltpu.VMEM((1024, 64), F32),
        pltpu.SemaphoreType.DMA,
        pltpu.SemaphoreType.DMA,
    ],
)
def _k1(hn2, fsd2, z64, out2,
        acc_s, idxg, idxs, rows, semg, sems):
    cid, sid = _wid_axes()
    s0 = sid * NSL

    stg = rows.at[pl.ds(0, NSL), :]
    pltpu.sync_copy(z64, stg)
    pltpu.sync_copy(stg, acc_s.at[pl.ds(s0, NSL)])
    plsc.subcore_barrier()

    def body(k, _):
        c = sid + 16 * k
        r0 = c * 8
        pltpu.sync_copy(fsd2.at[pl.ds(cid * 384 + r0, 8)], idxg)
        pltpu.sync_copy(fsd2.at[pl.ds((1 - cid) * 384 + r0, 8)], idxs)

        off = cid * NPAD

        @plsc.parallel_loop(0, 64, 1, unroll=8)
        def _(i):
            r = i // 8
            cc = (i % 8) * 16
            idxg[r, pl.ds(cc, 16)] = idxg[r, pl.ds(cc, 16)] + off

        _fire_drain([
            pltpu.async_copy(hn2.at[idxg.at[j]],
                             rows.at[pl.ds(j * 128, 128)], semg)
            for j in range(8)
        ])
        _fire_drain([
            pltpu.async_copy(rows.at[pl.ds(j * 128, 128)],
                             acc_s.at[idxs.at[j]], sems, add=True)
            for j in range(8)
        ])
        return 0

    lax.fori_loop(0, 3, body, 0)
    plsc.subcore_barrier()

    pltpu.sync_copy(acc_s.at[pl.ds(s0, NSL)], stg)
    pltpu.sync_copy(stg, out2.at[pl.ds(cid * NPAD + s0, NSL)])


@functools.partial(
    pl.kernel,
    out_type=[jax.ShapeDtypeStruct((2 * NPAD, 32), F32)],
    mesh=_mesh,
    compiler_params=pltpu.CompilerParams(use_tc_tiling_on_sc=False),
    scratch_types=[
        pltpu.VMEM_SHARED((NPAD, 32), F32),
        pltpu.VMEM((8, 128), I32),
        pltpu.VMEM((8, 128), I32),
        pltpu.VMEM((8, 128), I32),
        pltpu.VMEM((8, 128), I32),
        pltpu.VMEM((1024, 32), F32),
        pltpu.VMEM((1024, 32), F32),
        pltpu.SemaphoreType.DMA,
        pltpu.SemaphoreType.DMA,
        pltpu.SemaphoreType.DMA,
    ],
)
def _k2(hc2, pci2, pni2, z2d, out2,
        acc_s, idxg0, idxg1, idxs0, idxs1, rows0, rows1,
        semg, sems0, sems1):
    cid, sid = _wid_axes()
    s0 = sid * NSL
    goff = cid * CP

    pltpu.sync_copy(z2d.at[pl.ds(0, 1024)], rows0)
    pltpu.sync_copy(z2d.at[pl.ds(0, 1024)], rows1)
    pltpu.sync_copy(rows0.at[pl.ds(0, NSL), :], acc_s.at[pl.ds(s0, NSL)])
    plsc.subcore_barrier()

    def drain_scat(sems):
        pltpu.make_async_copy(hc2.at[pl.ds(0, 1024)], rows0, sems).wait()

    pltpu.sync_copy(pni2.at[pl.ds(0, 8)], idxs0)
    pltpu.sync_copy(pni2.at[pl.ds(0, 8)], idxs1)
    for j in range(8):
        pltpu.async_copy(rows0.at[pl.ds(j * 128, 128)],
                         acc_s.at[idxs0.at[j]], sems0, add=True)
    for j in range(8):
        pltpu.async_copy(rows1.at[pl.ds(j * 128, 128)],
                         acc_s.at[idxs1.at[j]], sems1, add=True)

    def do_chunk(k, idxg, idxs, rows, sems):
        c = sid + 16 * k
        r0 = c * 8
        pltpu.sync_copy(pci2.at[pl.ds(r0, 8)], idxg)
        pltpu.sync_copy(pni2.at[pl.ds(r0, 8)], idxs)

        @plsc.parallel_loop(0, 64, 1, unroll=8)
        def _(i):
            r = i // 8
            cc = (i % 8) * 16
            idxg[r, pl.ds(cc, 16)] = idxg[r, pl.ds(cc, 16)] + goff

        drain_scat(sems)
        for j in range(8):
            pltpu.async_copy(hc2.at[idxg.at[j]],
                             rows.at[pl.ds(j * 128, 128)], semg)
        pltpu.make_async_copy(hc2.at[pl.ds(0, 1024)], rows0, semg).wait()
        for j in range(8):
            pltpu.async_copy(rows.at[pl.ds(j * 128, 128)],
                             acc_s.at[idxs.at[j]], sems, add=True)

    def body(k, _):
        do_chunk(2 * k, idxg0, idxs0, rows0, sems0)
        do_chunk(2 * k + 1, idxg1, idxs1, rows1, sems1)
        return 0

    lax.fori_loop(0, 25, body, 0)
    drain_scat(sems0)
    drain_scat(sems1)
    plsc.subcore_barrier()

    stgo = rows0.at[pl.ds(0, NSL), :]
    pltpu.sync_copy(acc_s.at[pl.ds(s0, NSL)], stgo)
    pltpu.sync_copy(stgo, out2.at[pl.ds(cid * NPAD + s0, NSL)])


@functools.partial(
    pl.kernel,
    out_type=[jax.ShapeDtypeStruct((CP, 16), F32) for _ in range(4)],
    mesh=_mesh,
    compiler_params=pltpu.CompilerParams(use_tc_tiling_on_sc=False),
    scratch_types=[
        pltpu.VMEM_SHARED((CP, 16), F32),
        pltpu.VMEM_SHARED((NPAD, 16), F32),
        pltpu.VMEM((8, 128), I32),
        pltpu.VMEM((8, 128), I32),
        pltpu.VMEM((8, 128), I32),
        pltpu.VMEM((8, 128), I32),
        pltpu.VMEM((1024, 16), F32),
        pltpu.VMEM((1024, 16), F32),
        pltpu.VMEM((1024, 16), F32),
        pltpu.VMEM((1024, 16), F32),
        pltpu.SemaphoreType.DMA,
        pltpu.SemaphoreType.DMA,
        pltpu.SemaphoreType.DMA,
        pltpu.SemaphoreType.DMA,
    ],
)
def _k3(hv0, hv1, hv2, hv3, he0, he1, he2, he3, pci2, pni2, z16,
        out0, out1, out2, out3,
        acc_s, tab_s, idxg0, idxg1, idxs0, idxs1, rows0, rows1, heb0, heb1,
        semg, semh, sems0, sems1):
    cid, sid = _wid_axes()
    c0 = sid * CSL
    s0 = sid * NSL

    def one_pass(hvq, heq, outq):
        stg = rows0.at[pl.ds(0, NSL), :]
        pltpu.sync_copy(hvq.at[pl.ds(s0, NSL)], stg)
        pltpu.sync_copy(stg, tab_s.at[pl.ds(s0, NSL)])
        pltpu.sync_copy(z16.at[pl.ds(0, 1024)], rows0)
        pltpu.sync_copy(z16.at[pl.ds(0, 1024)], rows1)
        for off, n in ((0, 1024), (1024, 1024), (2048, 1024), (3072, 56)):
            stgz = rows0.at[pl.ds(0, n), :]
            pltpu.sync_copy(stgz, acc_s.at[pl.ds(c0 + off, n)])
        plsc.subcore_barrier()

        def drain_scat(sems):
            pltpu.make_async_copy(heq.at[pl.ds(0, 1024), :], heb0,
                                  sems).wait()

        pltpu.sync_copy(pci2.at[pl.ds(0, 8)], idxs0)
        pltpu.sync_copy(pci2.at[pl.ds(0, 8)], idxs1)
        for j in range(8):
            pltpu.async_copy(rows0.at[pl.ds(j * 128, 128), :],
                             acc_s.at[idxs0.at[j]], sems0, add=True)
        for j in range(8):
            pltpu.async_copy(rows1.at[pl.ds(j * 128, 128), :],
                             acc_s.at[idxs1.at[j]], sems1, add=True)

        def do_chunk(k, idxg, idxs, rows, heb, sems):
            c = sid + 16 * k
            r0 = c * 8
            pltpu.sync_copy(pni2.at[pl.ds(r0, 8)], idxg)
            pltpu.sync_copy(pci2.at[pl.ds(r0, 8)], idxs)
            hed = pltpu.async_copy(heq.at[pl.ds(c * 1024, 1024), :], heb,
                                   semh)
            drain_scat(sems)
            _fire_drain([
                pltpu.async_copy(tab_s.at[idxg.at[j]],
                                 rows.at[pl.ds(j * 128, 128)], semg)
                for j in range(8)
            ])
            hed.wait()

            @plsc.parallel_loop(0, 1024, 1, unroll=8)
            def _(q):
                rows[q, pl.ds(0, 16)] = (heb[q, pl.ds(0, 16)] *
                                         rows[q, pl.ds(0, 16)])

            for j in range(8):
                pltpu.async_copy(rows.at[pl.ds(j * 128, 128), :],
                                 acc_s.at[idxs.at[j]], sems, add=True)

        def body(j, _):
            do_chunk(2 * j, idxg0, idxs0, rows0, heb0, sems0)
            do_chunk(2 * j + 1, idxg1, idxs1, rows1, heb1, sems1)
            return 0

        lax.fori_loop(0, 25, body, 0)
        drain_scat(sems0)
        drain_scat(sems1)
        plsc.subcore_barrier()

        for off, n in ((0, 1024), (1024, 1024), (2048, 1024), (3072, 56)):
            stgo = rows0.at[pl.ds(0, n), :]
            pltpu.sync_copy(acc_s.at[pl.ds(c0 + off, n)], stgo)
            pltpu.sync_copy(stgo, outq.at[pl.ds(c0 + off, n)])
        plsc.subcore_barrier()

    @pl.when(cid == 0)
    def _():
        one_pass(hv0, he0, out0)
        one_pass(hv1, he1, out1)

    @pl.when(cid == 1)
    def _():
        one_pass(hv2, he2, out2)
        one_pass(hv3, he3, out3)


@functools.partial(
    pl.kernel,
    out_type=[
        jax.ShapeDtypeStruct((320, 128), F32),
        jax.ShapeDtypeStruct((320, 128), F32),
        jax.ShapeDtypeStruct((PINR, 128), F32),
        jax.ShapeDtypeStruct((PINR, 128), F32),
    ],
    mesh=_mesh,
    compiler_params=pltpu.CompilerParams(use_tc_tiling_on_sc=False),
    scratch_types=[
        pltpu.VMEM_SHARED((UF,), F32),
        pltpu.VMEM_SHARED((SF,), F32),
        pltpu.VMEM((16, 128), I32),
        pltpu.VMEM((16, 128), I32),
        pltpu.VMEM((16, 128), I32),
        pltpu.VMEM((16, 128), I32),
        pltpu.VMEM((16, 128), I32),
        pltpu.VMEM((16, 128), I32),
        pltpu.VMEM((16, 128), F32),
        pltpu.VMEM((16, 128), F32),
        pltpu.VMEM((16, 128), F32),
        pltpu.VMEM((16, 128), F32),
        pltpu.VMEM((SF // 16,), F32),
        pltpu.SemaphoreType.DMA,
    ],
)
def _k4(uf, scf, nn0_2, nn1_2, pni2, pci2, nd0, na0, pd0, pa0,
        u_s, sc_s, ia, ib, f0, f1, f2, f3, g0, g1, g2, g3, stb, sem):
    cid, sid = _wid_axes()
    pltpu.sync_copy(uf.at[pl.ds(sid * (UF // 16), UF // 16)],
                    stb.at[pl.ds(0, UF // 16)])
    pltpu.sync_copy(stb.at[pl.ds(0, UF // 16)],
                    u_s.at[pl.ds(sid * (UF // 16), UF // 16)])
    pltpu.sync_copy(scf.at[pl.ds(sid * (SF // 16), SF // 16)], stb)
    pltpu.sync_copy(stb, sc_s.at[pl.ds(sid * (SF // 16), SF // 16)])
    plsc.subcore_barrier()

    @pl.when(cid == 0)
    def _():
        def nbody(k, _):
            c = sid + 16 * k

            @pl.when(c < 40)
            def _():
                r0 = c * 8
                pltpu.sync_copy(nn0_2.at[pl.ds(r0, 8)], ia.at[pl.ds(0, 8)])
                pltpu.sync_copy(nn1_2.at[pl.ds(r0, 8)], ib.at[pl.ds(0, 8)])

                @plsc.parallel_loop(0, 64, 1, unroll=8)
                def _(i):
                    r = i // 8
                    cc = (i % 8) * 16
                    va = ia[r, pl.ds(cc, 16)] * 8
                    vb = ib[r, pl.ds(cc, 16)] * 8
                    f0[r, pl.ds(cc, 16)] = va
                    f1[r, pl.ds(cc, 16)] = vb + 1
                    f2[r, pl.ds(cc, 16)] = va + 2
                    f3[r, pl.ds(cc, 16)] = vb + 3

                _fire_drain(
                    [pltpu.async_copy(u_s.at[f0.at[j]], g0.at[j], sem)
                     for j in range(8)] +
                    [pltpu.async_copy(u_s.at[f1.at[j]], g1.at[j], sem)
                     for j in range(8)] +
                    [pltpu.async_copy(u_s.at[f2.at[j]], g2.at[j], sem)
                     for j in range(8)] +
                    [pltpu.async_copy(u_s.at[f3.at[j]], g3.at[j], sem)
                     for j in range(8)])

                @plsc.parallel_loop(0, 64, 1, unroll=8)
                def _(i):
                    r = i // 8
                    cc = (i % 8) * 16
                    g0[r, pl.ds(cc, 16)] = (g0[r, pl.ds(cc, 16)] +
                                            g1[r, pl.ds(cc, 16)])
                    g2[r, pl.ds(cc, 16)] = (g2[r, pl.ds(cc, 16)] +
                                            g3[r, pl.ds(cc, 16)])

                pltpu.sync_copy(g0.at[pl.ds(0, 8)], nd0.at[pl.ds(r0, 8)])
                pltpu.sync_copy(g2.at[pl.ds(0, 8)], na0.at[pl.ds(r0, 8)])
            return 0

        lax.fori_loop(0, 3, nbody, 0)

    def pbody(k, _):
        ci = sid + 16 * k

        @pl.when(ci < 200)
        def _():
            c = 2 * ci + cid
            r0 = c * 16
            pltpu.sync_copy(pni2.at[pl.ds(r0, 16)], ia)
            pltpu.sync_copy(pci2.at[pl.ds(r0, 16)], ib)

            @plsc.parallel_loop(0, 128, 1, unroll=8)
            def _(i):
                r = i // 8
                cc = (i % 8) * 16
                va = ia[r, pl.ds(cc, 16)] * 8
                vb = ib[r, pl.ds(cc, 16)] * 2
                f0[r, pl.ds(cc, 16)] = va + 4
                f1[r, pl.ds(cc, 16)] = va + 5
                f2[r, pl.ds(cc, 16)] = vb
                f3[r, pl.ds(cc, 16)] = vb + 1

            _fire_drain(
                [pltpu.async_copy(u_s.at[f0.at[j]], g0.at[j], sem)
                 for j in range(16)] +
                [pltpu.async_copy(u_s.at[f1.at[j]], g1.at[j], sem)
                 for j in range(16)] +
                [pltpu.async_copy(sc_s.at[f2.at[j]], g2.at[j], sem)
                 for j in range(16)] +
                [pltpu.async_copy(sc_s.at[f3.at[j]], g3.at[j], sem)
                 for j in range(16)])

            @plsc.parallel_loop(0, 128, 1, unroll=8)
            def _(i):
                r = i // 8
                cc = (i % 8) * 16
                g0[r, pl.ds(cc, 16)] = (g0[r, pl.ds(cc, 16)] +
                                        g2[r, pl.ds(cc, 16)])
                g1[r, pl.ds(cc, 16)] = (g1[r, pl.ds(cc, 16)] +
                                        g3[r, pl.ds(cc, 16)])

            pltpu.sync_copy(g0, pd0.at[pl.ds(r0, 16)])
            pltpu.sync_copy(g1, pa0.at[pl.ds(r0, 16)])
        return 0

    lax.fori_loop(0, 13, pbody, 0)


def _t1a_body(x_ref, cnt_ref, w_ref, b_ref, lo_ref, hi_ref):
    h = jnp.tanh(jnp.dot(x_ref[...], w_ref[...],
                         preferred_element_type=F32) + b_ref[...])
    h = h * _rs(cnt_ref[...])
    lo_ref[...] = h[:, :32]
    hi_ref[...] = h[:, 32:]


def _t1b_body(x_ref, cnt3_ref, w_ref, b_ref, cw_ref, cb_ref,
              hs_ref, hd_ref, lo_ref):
    hn = jnp.tanh(jnp.dot(x_ref[...], w_ref[...],
                          preferred_element_type=F32) + b_ref[...])
    cnt3 = cnt3_ref[...]
    hs_ref[...] = hn * _rs(cnt3[:, 2:3])
    hd_ref[...] = hn * _rs(cnt3[:, 1:2])
    hv = jnp.dot(hn, cw_ref[...], preferred_element_type=F32) + cb_ref[...]
    lo_ref[...] = hv


def _t1c_body(x_ref, w1_ref, b1_ref, w2_ref, b2_ref, w3_ref, b3_ref, wsp_ref,
              q0_ref, q1_ref, q2_ref, q3_ref, sp_ref):
    hp = jnp.tanh(jnp.dot(x_ref[...], w1_ref[...],
                          preferred_element_type=F32) + b1_ref[...])
    t = _ssp(jnp.dot(hp, w2_ref[...], preferred_element_type=F32) + b2_ref[...])
    he = _ssp(jnp.dot(t, w3_ref[...], preferred_element_type=F32) + b3_ref[...])
    q0_ref[...] = he[:, :16]
    q1_ref[...] = he[:, 16:32]
    q2_ref[...] = he[:, 32:48]
    q3_ref[...] = he[:, 48:]
    sp_ref[...] = jnp.dot(hp, wsp_ref[...], preferred_element_type=F32)


def _t2a_body(a1l_ref, a1h_ref, af_ref, as_ref, cnt3_ref,
              wp_ref, bp_ref, wf_ref, bf_ref, ws_ref, bs_ref, w6_ref, b6_ref,
              u_ref):
    cnt3 = cnt3_ref[...]
    a1 = jnp.concatenate([a1l_ref[...], a1h_ref[...]], axis=-1)
    op = jnp.dot(a1 * _rs(cnt3[:, 0:1]), wp_ref[...],
                 preferred_element_type=F32) + bp_ref[...]
    of = jnp.dot(af_ref[...] * _rs(cnt3[:, 1:2]), wf_ref[...],
                 preferred_element_type=F32) + bf_ref[...]
    os_ = jnp.dot(as_ref[...] * _rs(cnt3[:, 2:3]), ws_ref[...],
                  preferred_element_type=F32) + bs_ref[...]
    h = jnp.maximum(jnp.maximum(op, of), os_)
    u_ref[...] = (jnp.dot(h, w6_ref[...], preferred_element_type=F32) +
                  b6_ref[...])


def _t2b_body(a40_ref, a41_ref, a42_ref, a43_ref, wo_ref, bo_ref, wsc_ref,
              sc_ref):
    a4 = jnp.concatenate([a40_ref[...], a41_ref[...], a42_ref[...],
                          a43_ref[...]], axis=-1)
    h = _ssp(jnp.dot(a4, wo_ref[...], preferred_element_type=F32) + bo_ref[...])
    sc_ref[...] = jnp.dot(h, wsc_ref[...], preferred_element_type=F32)


def _t3_body(pd_ref, pa_ref, sd_ref, sa_ref, dis_ref, ang_ref):
    dis_ref[...] = _softplus(pd_ref[...] + sd_ref[...])
    ang_ref[...] = pa_ref[...] + sa_ref[...]


def _t3b_body(nd_ref, dis_ref):
    dis_ref[...] = _softplus(nd_ref[...])


def _full(shape):
    nd = len(shape)
    return pl.BlockSpec(shape, lambda *_: (0,) * nd)


def kernel(cell_feat, net_feat, pin_feat, pin_cell_idx, pin_net_idx,
           father_src, father_dst, net_net_pair, params):
    p = params
    pci = pin_cell_idx.astype(I32)
    pni = pin_net_idx.astype(I32)
    fs = father_src.astype(I32)
    fd = father_dst.astype(I32)
    nn0 = net_net_pair[:, 0].astype(I32)
    nn1 = net_net_pair[:, 1].astype(I32)

    dead_n = 10048 + (jnp.arange(NNP - N_NN, dtype=I32) % 64)
    dead_f = 10048 + (jnp.arange(NNF - N_NN, dtype=I32) % 64)
    dead_np = 10048 + (jnp.arange(NPP - N_PIN, dtype=I32) % 64)
    dead_cp = 50000 + (jnp.arange(NPP - N_PIN, dtype=I32) % 48)
    pci2 = jnp.concatenate([pci, dead_cp]).reshape(PINR, 128)
    pni2 = jnp.concatenate([pni, dead_np]).reshape(PINR, 128)
    fs2 = jnp.concatenate([fs, dead_f]).reshape(384, 128)
    fd2 = jnp.concatenate([fd, dead_f]).reshape(384, 128)
    nn0_2 = jnp.concatenate([nn0, dead_n]).reshape(320, 128)
    nn1_2 = jnp.concatenate([nn1, dead_n]).reshape(320, 128)

    ones_h = jnp.ones((16, 128), F32)
    z1 = jnp.zeros((CSL,), F32)
    z2d = jnp.zeros((CSL, 32), F32)
    z16 = jnp.zeros((CSL, 16), F32)
    z64 = jnp.zeros((NSL, 64), F32)

    cell_p = jnp.pad(cell_feat, ((0, CP - N_CELL), (0, 0)))
    net_p = jnp.pad(net_feat, ((0, NPAD - N_NET), (0, 0)))
    pin_p = jnp.pad(pin_feat, ((0, NPP - N_PIN), (0, 0)))

    wd, wa = p["net_dis"]["W"][:, 0], p["net_angle"]["W"][:, 0]
    wpd, wpa = p["pin_dis"]["W"][:, 0], p["pin_angle"]["W"][:, 0]
    zc = jnp.zeros((64,), F32)
    w6 = jnp.stack([wd[:64], wd[64:], wa[:64], wa[64:], wpd[:64], wpa[:64],
                    zc, zc], axis=-1)
    e = jnp.eye(8, dtype=F32)
    b6 = (e[0] * p["net_dis"]["b"][0] + e[2] * p["net_angle"]["b"][0] +
          e[4] * p["pin_dis"]["b"][0] + e[5] * p["pin_angle"]["b"][0])
    b6 = b6.reshape(1, 8)
    wsp = jnp.stack([wpd[64:80], wpa[64:80]], axis=-1)
    wsc = jnp.stack([wpd[80:], wpa[80:]], axis=-1)

    cc, cn, cf, cd = _k0(pci2, pni2, fs2, fd2, ones_h, z1)
    cnt3 = jnp.stack([cn, cd, cf], axis=-1)

    hc_lo, hc_hi = pl.pallas_call(
        _t1a_body,
        grid=(23,),
        in_specs=[
            pl.BlockSpec((2176, 16), lambda i: (i, 0)),
            pl.BlockSpec((2176, 1), lambda i: (i, 0)),
            _full((16, 64)),
            _full((1, 64)),
        ],
        out_specs=[
            pl.BlockSpec((2176, 32), lambda i: (i, 0)),
            pl.BlockSpec((2176, 32), lambda i: (i, 0)),
        ],
        out_shape=[
            jax.ShapeDtypeStruct((CP, 32), F32),
            jax.ShapeDtypeStruct((CP, 32), F32),
        ],
    )(cell_p, cc.reshape(CP, 1), p["cell_lin"]["W"],
      p["cell_lin"]["b"].reshape(1, 64))

    hn_src, hn_dst, hv = pl.pallas_call(
        _t1b_body,
        grid=(8,),
        in_specs=[
            pl.BlockSpec((1264, 8), lambda i: (i, 0)),
            pl.BlockSpec((1264, 3), lambda i: (i, 0)),
            _full((8, 64)),
            _full((1, 64)),
            _full((64, 64)),
            _full((1, 64)),
        ],
        out_specs=[
            pl.BlockSpec((1264, 64), lambda i: (i, 0)),
            pl.BlockSpec((1264, 64), lambda i: (i, 0)),
            pl.BlockSpec((1264, 64), lambda i: (i, 0)),
        ],
        out_shape=[
            jax.ShapeDtypeStruct((NPAD, 64), F32),
            jax.ShapeDtypeStruct((NPAD, 64), F32),
            jax.ShapeDtypeStruct((NPAD, 64), F32),
        ],
    )(net_p, cnt3, p["net_lin"]["W"], p["net_lin"]["b"].reshape(1, 64),
      p["cf_node"]["W"], p["cf_node"]["b"].reshape(1, 64))

    he0, he1, he2, he3, s_pin = pl.pallas_call(
        _t1c_body,
        grid=(200,),
        in_specs=[
            pl.BlockSpec((4096, 8), lambda i: (i, 0)),
            _full((8, 16)),
            _full((1, 16)),
            _full((16, 64)),
            _full((1, 64)),
            _full((64, 64)),
            _full((1, 64)),
            _full((16, 2)),
        ],
        out_specs=[
            pl.BlockSpec((4096, 16), lambda i: (i, 0)),
            pl.BlockSpec((4096, 16), lambda i: (i, 0)),
            pl.BlockSpec((4096, 16), lambda i: (i, 0)),
            pl.BlockSpec((4096, 16), lambda i: (i, 0)),
            pl.BlockSpec((4096, 2), lambda i: (i, 0)),
        ],
        out_shape=[
            jax.ShapeDtypeStruct((NPP, 16), F32),
            jax.ShapeDtypeStruct((NPP, 16), F32),
            jax.ShapeDtypeStruct((NPP, 16), F32),
            jax.ShapeDtypeStruct((NPP, 16), F32),
            jax.ShapeDtypeStruct((NPP, 2), F32),
        ],
    )(pin_p, p["pin_lin"]["W"], p["pin_lin"]["b"].reshape(1, 16),
      p["cf_edge1"]["W"], p["cf_edge1"]["b"].reshape(1, 64),
      p["cf_edge2"]["W"], p["cf_edge2"]["b"].reshape(1, 64), wsp)

    fsd2 = jnp.concatenate([fs2, fd2], axis=0)
    hn2 = jnp.concatenate([hn_src, hn_dst], axis=0)
    out2 = _k1(hn2, fsd2, z64)[0]
    accf, accs = out2[:NPAD], out2[NPAD:]
    hc2 = jnp.concatenate([hc_lo, hc_hi], axis=0)
    outk2 = _k2(hc2, pci2, pni2, z2d)[0]
    acc1_lo, acc1_hi = outk2[:NPAD], outk2[NPAD:]
    hv0, hv1, hv2, hv3 = (hv[:, :16], hv[:, 16:32], hv[:, 32:48], hv[:, 48:])
    a40, a41, a42, a43 = _k3(hv0, hv1, hv2, hv3, he0, he1, he2, he3,
                             pci2, pni2, z16)

    u = pl.pallas_call(
        _t2a_body,
        grid=(4,),
        in_specs=[
            pl.BlockSpec((2528, 32), lambda i: (i, 0)),
            pl.BlockSpec((2528, 32), lambda i: (i, 0)),
            pl.BlockSpec((2528, 64), lambda i: (i, 0)),
            pl.BlockSpec((2528, 64), lambda i: (i, 0)),
            pl.BlockSpec((2528, 3), lambda i: (i, 0)),
            _full((64, 64)), _full((1, 64)),
            _full((64, 64)), _full((1, 64)),
            _full((64, 64)), _full((1, 64)),
            _full((64, 8)), _full((1, 8)),
        ],
        out_specs=[pl.BlockSpec((2528, 8), lambda i: (i, 0))],
        out_shape=[jax.ShapeDtypeStruct((NPAD, 8), F32)],
    )(acc1_lo, acc1_hi, accf, accs, cnt3,
      p["gc_pins"]["W"], p["gc_pins"]["b"].reshape(1, 64),
      p["gc_father"]["W"], p["gc_father"]["b"].reshape(1, 64),
      p["gc_son"]["W"], p["gc_son"]["b"].reshape(1, 64),
      w6, b6)[0]

    s_cell = pl.pallas_call(
        _t2b_body,
        grid=(23,),
        in_specs=[
            pl.BlockSpec((2176, 16), lambda i: (i, 0)),
            pl.BlockSpec((2176, 16), lambda i: (i, 0)),
            pl.BlockSpec((2176, 16), lambda i: (i, 0)),
            pl.BlockSpec((2176, 16), lambda i: (i, 0)),
            _full((64, 64)), _full((1, 64)), _full((64, 2)),
        ],
        out_specs=[pl.BlockSpec((2176, 2), lambda i: (i, 0))],
        out_shape=[jax.ShapeDtypeStruct((CP, 2), F32)],
    )(a40, a41, a42, a43, p["cf_out"]["W"], p["cf_out"]["b"].reshape(1, 64),
      wsc)[0]

    nd0, na0, pd0, pa0 = _k4(u.reshape(UF), s_cell.reshape(SF),
                             nn0_2, nn1_2, pni2, pci2)

    net_dis2 = pl.pallas_call(
        _t3b_body,
        in_specs=[_full((320, 128))],
        out_specs=[_full((320, 128))],
        out_shape=[jax.ShapeDtypeStruct((320, 128), F32)],
    )(nd0)[0]

    sd = s_pin[:, 0].reshape(PINR, 128)
    sa = s_pin[:, 1].reshape(PINR, 128)
    dis2, ang2 = pl.pallas_call(
        _t3_body,
        grid=(8,),
        in_specs=[pl.BlockSpec((800, 128), lambda i: (i, 0))] * 4,
        out_specs=[pl.BlockSpec((800, 128), lambda i: (i, 0))] * 2,
        out_shape=[
            jax.ShapeDtypeStruct((PINR, 128), F32),
            jax.ShapeDtypeStruct((PINR, 128), F32),
        ],
    )(pd0, pa0, sd, sa)

    return (net_dis2.reshape(-1)[:N_NN], na0.reshape(-1)[:N_NN],
            dis2.reshape(-1)[:N_PIN], ang2.reshape(-1)[:N_PIN])
